# baseline (speedup 1.0000x reference)
import functools

import jax
import jax.numpy as jnp
from jax import lax
from jax.experimental import pallas as pl
from jax.experimental.pallas import tpu as pltpu
from jax.experimental.pallas import tpu_sc as plsc

_NUM_GROUPS = 256
_GROUP_SIZE = 32
_HIDDEN = 384
_EPS = 1e-5


def _fps_body(pts_ref, f0_ref, cen_ref):
    X = pts_ref[0]
    Y = pts_ref[1]
    Z = pts_ref[2]
    B, N = X.shape
    col = jax.lax.broadcasted_iota(jnp.int32, (B, N), 1)
    colc = jax.lax.broadcasted_iota(jnp.int32, (B, _NUM_GROUPS), 1)
    cen_ref[...] = jnp.zeros_like(cen_ref)

    def body(i, carry):
        dist, f = carry
        onehot = col == f
        cx = jnp.sum(jnp.where(onehot, X, 0.0), axis=1, keepdims=True)
        cy = jnp.sum(jnp.where(onehot, Y, 0.0), axis=1, keepdims=True)
        cz = jnp.sum(jnp.where(onehot, Z, 0.0), axis=1, keepdims=True)
        sel = colc == i
        cen_ref[0] = jnp.where(sel, cx, cen_ref[0])
        cen_ref[1] = jnp.where(sel, cy, cen_ref[1])
        cen_ref[2] = jnp.where(sel, cz, cen_ref[2])
        d = (X - cx) ** 2 + (Y - cy) ** 2 + (Z - cz) ** 2
        dist = jnp.minimum(dist, d)
        m = jnp.max(dist, axis=1, keepdims=True)
        f = jnp.min(jnp.where(dist == m, col, N), axis=1, keepdims=True)
        return dist, f

    dist0 = jnp.full((B, N), 1e10, dtype=jnp.float32)
    f0 = f0_ref[...]
    jax.lax.fori_loop(0, _NUM_GROUPS, body, (dist0, f0))


def _fps(points):
    B, N, _ = points.shape
    pts_t = points.transpose(2, 0, 1)
    f0 = jax.random.randint(jax.random.key(42), (B,), 0, N).astype(jnp.int32)
    cen_t = pl.pallas_call(
        _fps_body,
        out_shape=jax.ShapeDtypeStruct((3, B, _NUM_GROUPS), jnp.float32),
    )(pts_t, f0.reshape(B, 1))
    return cen_t


def _l1_body(w1t_ref, b1_ref, g_ref, h1_ref, ssum_ref, ssq_ref):
    i = pl.program_id(0)
    h = jnp.dot(w1t_ref[...], g_ref[...], preferred_element_type=jnp.float32)
    h = h + b1_ref[...]
    h1_ref[...] = h

    @pl.when(i == 0)
    def _():
        ssum_ref[...] = jnp.zeros_like(ssum_ref)
        ssq_ref[...] = jnp.zeros_like(ssq_ref)

    ssum_ref[...] += jnp.sum(h, axis=1, keepdims=True)
    ssq_ref[...] += jnp.sum(h * h, axis=1, keepdims=True)


def _l2_body(w2t_ref, b2_ref, g1_ref, be1_ref, s1_ref, q1_ref, h1_ref,
             h2_ref, ssum_ref, ssq_ref, *, n_total):
    i = pl.program_id(0)
    mean = s1_ref[...] / n_total
    var = q1_ref[...] / n_total - mean * mean
    s = g1_ref[...] * jax.lax.rsqrt(var + _EPS)
    t = be1_ref[...] - mean * s
    a = jax.nn.relu(s * h1_ref[...] + t)
    h = jnp.dot(w2t_ref[...], a, preferred_element_type=jnp.float32)
    h = h + b2_ref[...]
    h2_ref[...] = h

    @pl.when(i == 0)
    def _():
        ssum_ref[...] = jnp.zeros_like(ssum_ref)
        ssq_ref[...] = jnp.zeros_like(ssq_ref)

    ssum_ref[...] += jnp.sum(h, axis=1, keepdims=True)
    ssq_ref[...] += jnp.sum(h * h, axis=1, keepdims=True)


def _l3_body(w3t_ref, b3_ref, g2_ref, be2_ref, s2_ref, q2_ref, h2_ref,
             mx_ref, mn_ref, ssum_ref, ssq_ref, *, n_total):
    j = pl.program_id(0)
    mean = s2_ref[...] / n_total
    var = q2_ref[...] / n_total - mean * mean
    s = g2_ref[...] * jax.lax.rsqrt(var + _EPS)
    t = be2_ref[...] - mean * s
    a = jax.nn.relu(s * h2_ref[...] + t)
    h = jnp.dot(w3t_ref[...], a, preferred_element_type=jnp.float32)
    h = h + b3_ref[...]

    @pl.when(j == 0)
    def _():
        ssum_ref[...] = jnp.zeros_like(ssum_ref)
        ssq_ref[...] = jnp.zeros_like(ssq_ref)
        mx_ref[...] = jnp.full_like(mx_ref, -jnp.inf)
        mn_ref[...] = jnp.full_like(mn_ref, jnp.inf)

    ssum_ref[...] += jnp.sum(h, axis=1, keepdims=True)
    ssq_ref[...] += jnp.sum(h * h, axis=1, keepdims=True)
    mx_ref[...] = jnp.maximum(mx_ref[...], h)
    mn_ref[...] = jnp.minimum(mn_ref[...], h)


def _fin_body(g3_ref, be3_ref, s3_ref, q3_ref, mx_ref, mn_ref, tok_ref, *, n_total):
    mean = s3_ref[...] / n_total
    var = q3_ref[...] / n_total - mean * mean
    s = g3_ref[...] * jax.lax.rsqrt(var + _EPS)
    t = be3_ref[...] - mean * s
    picked = jnp.where(s >= 0.0, mx_ref[...], mn_ref[...])
    tok_ref[...] = picked * s + t


def _pointnet(G, W1, b1, g1, be1, W2, b2, g2, be2, W3, b3, g3, be3):
    S = G.shape[1]
    NG = S // _GROUP_SIZE
    cvec = lambda v: v.reshape(-1, 1)

    LBLK = 8192
    nblk = S // LBLK
    h1, s1, q1 = pl.pallas_call(
        _l1_body,
        grid=(nblk,),
        in_specs=[
            pl.BlockSpec((64, 3), lambda i: (0, 0)),
            pl.BlockSpec((64, 1), lambda i: (0, 0)),
            pl.BlockSpec((3, LBLK), lambda i: (0, i)),
        ],
        out_specs=[
            pl.BlockSpec((64, LBLK), lambda i: (0, i)),
            pl.BlockSpec((64, 1), lambda i: (0, 0)),
            pl.BlockSpec((64, 1), lambda i: (0, 0)),
        ],
        out_shape=[
            jax.ShapeDtypeStruct((64, S), jnp.float32),
            jax.ShapeDtypeStruct((64, 1), jnp.float32),
            jax.ShapeDtypeStruct((64, 1), jnp.float32),
        ],
    )(W1.T, cvec(b1), G)

    h2, s2, q2 = pl.pallas_call(
        functools.partial(_l2_body, n_total=float(S)),
        grid=(nblk,),
        in_specs=[
            pl.BlockSpec((128, 64), lambda i: (0, 0)),
            pl.BlockSpec((128, 1), lambda i: (0, 0)),
            pl.BlockSpec((64, 1), lambda i: (0, 0)),
            pl.BlockSpec((64, 1), lambda i: (0, 0)),
            pl.BlockSpec((64, 1), lambda i: (0, 0)),
            pl.BlockSpec((64, 1), lambda i: (0, 0)),
            pl.BlockSpec((64, LBLK), lambda i: (0, i)),
        ],
        out_specs=[
            pl.BlockSpec((128, LBLK), lambda i: (0, i)),
            pl.BlockSpec((128, 1), lambda i: (0, 0)),
            pl.BlockSpec((128, 1), lambda i: (0, 0)),
        ],
        out_shape=[
            jax.ShapeDtypeStruct((128, S), jnp.float32),
            jax.ShapeDtypeStruct((128, 1), jnp.float32),
            jax.ShapeDtypeStruct((128, 1), jnp.float32),
        ],
    )(W2.T, cvec(b2), cvec(g1), cvec(be1), s1, q1, h1)

    mx, mn, s3, q3 = pl.pallas_call(
        functools.partial(_l3_body, n_total=float(S)),
        grid=(_GROUP_SIZE,),
        in_specs=[
            pl.BlockSpec((_HIDDEN, 128), lambda j: (0, 0)),
            pl.BlockSpec((_HIDDEN, 1), lambda j: (0, 0)),
            pl.BlockSpec((128, 1), lambda j: (0, 0)),
            pl.BlockSpec((128, 1), lambda j: (0, 0)),
            pl.BlockSpec((128, 1), lambda j: (0, 0)),
            pl.BlockSpec((128, 1), lambda j: (0, 0)),
            pl.BlockSpec((128, NG), lambda j: (0, j)),
        ],
        out_specs=[
            pl.BlockSpec((_HIDDEN, NG), lambda j: (0, 0)),
            pl.BlockSpec((_HIDDEN, NG), lambda j: (0, 0)),
            pl.BlockSpec((_HIDDEN, 1), lambda j: (0, 0)),
            pl.BlockSpec((_HIDDEN, 1), lambda j: (0, 0)),
        ],
        out_shape=[
            jax.ShapeDtypeStruct((_HIDDEN, NG), jnp.float32),
            jax.ShapeDtypeStruct((_HIDDEN, NG), jnp.float32),
            jax.ShapeDtypeStruct((_HIDDEN, 1), jnp.float32),
            jax.ShapeDtypeStruct((_HIDDEN, 1), jnp.float32),
        ],
    )(W3.T, cvec(b3), cvec(g2), cvec(be2), s2, q2, h2)

    tok_t = pl.pallas_call(
        functools.partial(_fin_body, n_total=float(S)),
        out_shape=jax.ShapeDtypeStruct((_HIDDEN, NG), jnp.float32),
    )(cvec(g3), cvec(be3), s3, q3, mx, mn)
    return tok_t


_NTILE = 32
_CPT = 64
_N = 8192
_NCHUNK = _N // 16


def _knn_sc_body(pts_hbm, cen_hbm, out_hbm,
                 X, Y, Z, CX, CY, CZ, d2i, subhist, histc, sel,
                 cua, cia, cub, cib, outbuf):
    NC = 2
    wid = lax.axis_index("s") * NC + lax.axis_index("c")
    b = wid // 4
    mb = (wid % 4) * _CPT

    pltpu.sync_copy(pts_hbm.at[pl.ds((b * 3 + 0) * _N, _N)], X)
    pltpu.sync_copy(pts_hbm.at[pl.ds((b * 3 + 1) * _N, _N)], Y)
    pltpu.sync_copy(pts_hbm.at[pl.ds((b * 3 + 2) * _N, _N)], Z)
    pltpu.sync_copy(cen_hbm.at[pl.ds((b * 3 + 0) * _NUM_GROUPS + mb, _CPT)],
                    CX.at[pl.ds(0, _CPT)])
    pltpu.sync_copy(cen_hbm.at[pl.ds((b * 3 + 1) * _NUM_GROUPS + mb, _CPT)],
                    CY.at[pl.ds(0, _CPT)])
    pltpu.sync_copy(cen_hbm.at[pl.ds((b * 3 + 2) * _NUM_GROUPS + mb, _CPT)],
                    CZ.at[pl.ds(0, _CPT)])

    iota16 = lax.iota(jnp.int32, 16)
    ones16 = jnp.ones((16,), jnp.int32)
    zeros16 = jnp.zeros((16,), jnp.int32)
    lanebase = iota16 * 256

    def zero_subhist():
        def z(t, c):
            subhist[pl.ds(t * 16, 16)] = zeros16
            return c
        lax.fori_loop(0, 256, z, 0)

    def reduce_hist():
        def red(t, c):
            s = zeros16
            for l in range(16):
                s = s + subhist[pl.ds(l * 256 + t * 16, 16)]
            histc[pl.ds(t * 16, 16)] = s
            return c
        lax.fori_loop(0, 16, red, 0)

    def find_threshold(k_rem):
        def th(t, carry):
            base, T = carry
            h = histc[pl.ds(t * 16, 16)]
            cums = plsc.cumsum(h) + base
            glane = iota16 + t * 16
            candT = jnp.min(jnp.where(cums >= k_rem, glane, 9999))
            return base + jnp.sum(h), jnp.minimum(T, candT)
        _, T = lax.fori_loop(0, 16, th, (jnp.int32(0), jnp.int32(9999)))
        return T

    def row(m, carry0):
        msplat = jnp.full((16,), m, jnp.int32)
        cxv = plsc.load_gather(CX, [msplat])
        cyv = plsc.load_gather(CY, [msplat])
        czv = plsc.load_gather(CZ, [msplat])

        zero_subhist()

        def p1(t, c):
            dx = X[pl.ds(t * 16, 16)] - cxv
            dy = Y[pl.ds(t * 16, 16)] - cyv
            dz = Z[pl.ds(t * 16, 16)] - czv
            d2 = dx * dx + dy * dy + dz * dz
            u = plsc.bitcast(d2, jnp.int32)
            d2i[pl.ds(t * 16, 16)] = u
            bkt = lax.shift_right_logical(u, 23)
            plsc.addupdate_scatter(subhist, [lanebase + bkt], ones16)
            return c
        lax.fori_loop(0, _NCHUNK, p1, 0)

        reduce_hist()
        T1 = find_threshold(jnp.int32(_GROUP_SIZE))

        def cp1(t, carry):
            selcnt, nout = carry
            u = d2i[pl.ds(t * 16, 16)]
            glane = iota16 + t * 16
            bkt = lax.shift_right_logical(u, 23)
            selm = bkt < T1
            eqm = bkt == T1
            plsc.store_compressed(sel.at[pl.ds(selcnt, 16)], glane, mask=selm)
            plsc.store_compressed(cua.at[pl.ds(nout, 16)], u, mask=eqm)
            plsc.store_compressed(cia.at[pl.ds(nout, 16)], glane, mask=eqm)
            return (selcnt + jnp.sum(selm.astype(jnp.int32)),
                    nout + jnp.sum(eqm.astype(jnp.int32)))
        selcnt, ncand = lax.fori_loop(0, _NCHUNK, cp1,
                                      (jnp.int32(0), jnp.int32(0)))

        bufs = [(cua, cia, cub, cib), (cub, cib, cua, cia), (cua, cia, cub, cib)]
        for (inu, ini, outu, outi), shift in zip(bufs, (15, 7, 0)):
            k_rem = _GROUP_SIZE - selcnt
            nch = (ncand + 15) >> 4
            zero_subhist()

            def h2(t, c, inu=inu, shift=shift, ncand=ncand):
                u = inu[pl.ds(t * 16, 16)]
                glane = iota16 + t * 16
                valid = glane < ncand
                bkt = lax.shift_right_logical(u, shift) & 0xFF
                plsc.addupdate_scatter(subhist, [lanebase + bkt], ones16,
                                       mask=valid)
                return c
            lax.fori_loop(0, nch, h2, 0)

            reduce_hist()
            T = find_threshold(k_rem)

            def cp(t, carry, inu=inu, ini=ini, outu=outu, outi=outi,
                   shift=shift, ncand=ncand, T=T):
                scnt, nout = carry
                u = inu[pl.ds(t * 16, 16)]
                ii = ini[pl.ds(t * 16, 16)]
                glane = iota16 + t * 16
                valid = glane < ncand
                bkt = lax.shift_right_logical(u, shift) & 0xFF
                selm = valid & (bkt < T)
                eqm = valid & (bkt == T)
                plsc.store_compressed(sel.at[pl.ds(scnt, 16)], ii, mask=selm)
                plsc.store_compressed(outu.at[pl.ds(nout, 16)], u, mask=eqm)
                plsc.store_compressed(outi.at[pl.ds(nout, 16)], ii, mask=eqm)
                return (scnt + jnp.sum(selm.astype(jnp.int32)),
                        nout + jnp.sum(eqm.astype(jnp.int32)))
            selcnt, ncand = lax.fori_loop(0, nch, cp, (selcnt, jnp.int32(0)))

        k_rem = _GROUP_SIZE - selcnt

        def fill(t, scnt):
            ii = cib[pl.ds(t * 16, 16)]
            glane = iota16 + t * 16
            maskf = glane < k_rem
            plsc.store_compressed(sel.at[pl.ds(scnt, 16)], ii, mask=maskf)
            return scnt + jnp.sum(maskf.astype(jnp.int32))
        selcnt = lax.fori_loop(0, 2, fill, selcnt)

        def gat(c, _):
            idxv = sel[pl.ds(c * 16, 16)]
            jv = iota16 + c * 16
            px = plsc.load_gather(X, [idxv]) - cxv
            py = plsc.load_gather(Y, [idxv]) - cyv
            pz = plsc.load_gather(Z, [idxv]) - czv
            zsp = jnp.zeros((16,), jnp.int32)
            plsc.store_scatter(outbuf, [zsp, jv, msplat], px)
            plsc.store_scatter(outbuf, [zsp + 1, jv, msplat], py)
            plsc.store_scatter(outbuf, [zsp + 2, jv, msplat], pz)
            return _
        lax.fori_loop(0, 2, gat, 0)
        return carry0

    lax.fori_loop(0, _CPT, row, 0)
    pltpu.sync_copy(outbuf, out_hbm.at[wid])


def _knn_sc(points, cen_t):
    B = points.shape[0]
    pts_flat = points.transpose(0, 2, 1).reshape(B * 3 * _N)
    cen_flat = cen_t.transpose(1, 0, 2).reshape(B * 3 * _NUM_GROUPS)
    mesh = plsc.VectorSubcoreMesh(core_axis_name="c", subcore_axis_name="s")
    f32, i32 = jnp.float32, jnp.int32
    out = pl.kernel(
        _knn_sc_body,
        mesh=mesh,
        out_type=jax.ShapeDtypeStruct((_NTILE, 3, _GROUP_SIZE, _CPT), f32),
        compiler_params=pltpu.CompilerParams(needs_layout_passes=False),
        scratch_types=[
            pltpu.VMEM((_N,), f32), pltpu.VMEM((_N,), f32),
            pltpu.VMEM((_N,), f32),
            pltpu.VMEM((128,), f32), pltpu.VMEM((128,), f32),
            pltpu.VMEM((128,), f32),
            pltpu.VMEM((_N,), i32),
            pltpu.VMEM((4096,), i32),
            pltpu.VMEM((256,), i32),
            pltpu.VMEM((48,), i32),
            pltpu.VMEM((_N + 16,), i32), pltpu.VMEM((_N + 16,), i32),
            pltpu.VMEM((_N + 16,), i32), pltpu.VMEM((_N + 16,), i32),
            pltpu.VMEM((3, _GROUP_SIZE, _CPT), f32),
        ],
    )(pts_flat, cen_flat)
    G = out.transpose(1, 2, 0, 3).reshape(3, _GROUP_SIZE * B * _NUM_GROUPS)
    return G


def kernel(points, W1, b1, g1, be1, W2, b2, g2, be2, W3, b3, g3, be3):
    B, N, _ = points.shape
    cen_t = _fps(points)
    centers = cen_t.transpose(1, 2, 0)

    G = _knn_sc(points, cen_t)

    tok_t = _pointnet(G, W1, b1, g1, be1, W2, b2, g2, be2, W3, b3, g3, be3)
    tokens = tok_t.T.reshape(B, _NUM_GROUPS, _HIDDEN)
    return (tokens, centers)


# speedup vs baseline: 1.0934x; 1.0934x over previous
import functools

import jax
import jax.numpy as jnp
from jax import lax
from jax.experimental import pallas as pl
from jax.experimental.pallas import tpu as pltpu
from jax.experimental.pallas import tpu_sc as plsc

_NUM_GROUPS = 256
_GROUP_SIZE = 32
_HIDDEN = 384
_EPS = 1e-5


def _fps_body(pts_ref, f0_ref, cen_ref):
    X = pts_ref[0]
    Y = pts_ref[1]
    Z = pts_ref[2]
    B, N = X.shape
    col = jax.lax.broadcasted_iota(jnp.int32, (B, N), 1)
    colc = jax.lax.broadcasted_iota(jnp.int32, (B, _NUM_GROUPS), 1)
    cen_ref[...] = jnp.zeros_like(cen_ref)

    def body(i, carry):
        dist, f = carry
        onehot = col == f
        cx = jnp.sum(jnp.where(onehot, X, 0.0), axis=1, keepdims=True)
        cy = jnp.sum(jnp.where(onehot, Y, 0.0), axis=1, keepdims=True)
        cz = jnp.sum(jnp.where(onehot, Z, 0.0), axis=1, keepdims=True)
        sel = colc == i
        cen_ref[0] = jnp.where(sel, cx, cen_ref[0])
        cen_ref[1] = jnp.where(sel, cy, cen_ref[1])
        cen_ref[2] = jnp.where(sel, cz, cen_ref[2])
        d = (X - cx) ** 2 + (Y - cy) ** 2 + (Z - cz) ** 2
        dist = jnp.minimum(dist, d)
        m = jnp.max(dist, axis=1, keepdims=True)
        f = jnp.min(jnp.where(dist == m, col, N), axis=1, keepdims=True)
        return dist, f

    dist0 = jnp.full((B, N), 1e10, dtype=jnp.float32)
    f0 = f0_ref[...]
    jax.lax.fori_loop(0, _NUM_GROUPS, body, (dist0, f0))


def _fps(points):
    B, N, _ = points.shape
    pts_t = points.transpose(2, 0, 1)
    f0 = jax.random.randint(jax.random.key(42), (B,), 0, N).astype(jnp.int32)
    cen_t = pl.pallas_call(
        _fps_body,
        out_shape=jax.ShapeDtypeStruct((3, B, _NUM_GROUPS), jnp.float32),
    )(pts_t, f0.reshape(B, 1))
    return cen_t


def _l1_body(w1t_ref, b1_ref, g_ref, h1_ref, ssum_ref, ssq_ref):
    i = pl.program_id(0)
    h = jnp.dot(w1t_ref[...], g_ref[...], preferred_element_type=jnp.float32)
    h = h + b1_ref[...]
    h1_ref[...] = h

    @pl.when(i == 0)
    def _():
        ssum_ref[...] = jnp.zeros_like(ssum_ref)
        ssq_ref[...] = jnp.zeros_like(ssq_ref)

    ssum_ref[...] += jnp.sum(h, axis=1, keepdims=True)
    ssq_ref[...] += jnp.sum(h * h, axis=1, keepdims=True)


def _l2_body(w2t_ref, b2_ref, g1_ref, be1_ref, s1_ref, q1_ref, h1_ref,
             h2_ref, ssum_ref, ssq_ref, *, n_total):
    i = pl.program_id(0)
    mean = s1_ref[...] / n_total
    var = q1_ref[...] / n_total - mean * mean
    s = g1_ref[...] * jax.lax.rsqrt(var + _EPS)
    t = be1_ref[...] - mean * s
    a = jax.nn.relu(s * h1_ref[...] + t)
    h = jnp.dot(w2t_ref[...], a, preferred_element_type=jnp.float32)
    h = h + b2_ref[...]
    h2_ref[...] = h

    @pl.when(i == 0)
    def _():
        ssum_ref[...] = jnp.zeros_like(ssum_ref)
        ssq_ref[...] = jnp.zeros_like(ssq_ref)

    ssum_ref[...] += jnp.sum(h, axis=1, keepdims=True)
    ssq_ref[...] += jnp.sum(h * h, axis=1, keepdims=True)


def _l3_body(w3t_ref, b3_ref, g2_ref, be2_ref, s2_ref, q2_ref, h2_ref,
             mx_ref, mn_ref, ssum_ref, ssq_ref, *, n_total):
    j = pl.program_id(0)
    mean = s2_ref[...] / n_total
    var = q2_ref[...] / n_total - mean * mean
    s = g2_ref[...] * jax.lax.rsqrt(var + _EPS)
    t = be2_ref[...] - mean * s
    a = jax.nn.relu(s * h2_ref[...] + t)
    h = jnp.dot(w3t_ref[...], a, preferred_element_type=jnp.float32)
    h = h + b3_ref[...]

    @pl.when(j == 0)
    def _():
        ssum_ref[...] = jnp.zeros_like(ssum_ref)
        ssq_ref[...] = jnp.zeros_like(ssq_ref)
        mx_ref[...] = jnp.full_like(mx_ref, -jnp.inf)
        mn_ref[...] = jnp.full_like(mn_ref, jnp.inf)

    ssum_ref[...] += jnp.sum(h, axis=1, keepdims=True)
    ssq_ref[...] += jnp.sum(h * h, axis=1, keepdims=True)
    mx_ref[...] = jnp.maximum(mx_ref[...], h)
    mn_ref[...] = jnp.minimum(mn_ref[...], h)


def _fin_body(g3_ref, be3_ref, s3_ref, q3_ref, mx_ref, mn_ref, tok_ref, *, n_total):
    mean = s3_ref[...] / n_total
    var = q3_ref[...] / n_total - mean * mean
    s = g3_ref[...] * jax.lax.rsqrt(var + _EPS)
    t = be3_ref[...] - mean * s
    picked = jnp.where(s >= 0.0, mx_ref[...], mn_ref[...])
    tok_ref[...] = picked * s + t


def _pointnet(G, W1, b1, g1, be1, W2, b2, g2, be2, W3, b3, g3, be3):
    S = G.shape[1]
    NG = S // _GROUP_SIZE
    cvec = lambda v: v.reshape(-1, 1)

    LBLK = 8192
    nblk = S // LBLK
    h1, s1, q1 = pl.pallas_call(
        _l1_body,
        grid=(nblk,),
        in_specs=[
            pl.BlockSpec((64, 3), lambda i: (0, 0)),
            pl.BlockSpec((64, 1), lambda i: (0, 0)),
            pl.BlockSpec((3, LBLK), lambda i: (0, i)),
        ],
        out_specs=[
            pl.BlockSpec((64, LBLK), lambda i: (0, i)),
            pl.BlockSpec((64, 1), lambda i: (0, 0)),
            pl.BlockSpec((64, 1), lambda i: (0, 0)),
        ],
        out_shape=[
            jax.ShapeDtypeStruct((64, S), jnp.float32),
            jax.ShapeDtypeStruct((64, 1), jnp.float32),
            jax.ShapeDtypeStruct((64, 1), jnp.float32),
        ],
    )(W1.T, cvec(b1), G)

    h2, s2, q2 = pl.pallas_call(
        functools.partial(_l2_body, n_total=float(S)),
        grid=(nblk,),
        in_specs=[
            pl.BlockSpec((128, 64), lambda i: (0, 0)),
            pl.BlockSpec((128, 1), lambda i: (0, 0)),
            pl.BlockSpec((64, 1), lambda i: (0, 0)),
            pl.BlockSpec((64, 1), lambda i: (0, 0)),
            pl.BlockSpec((64, 1), lambda i: (0, 0)),
            pl.BlockSpec((64, 1), lambda i: (0, 0)),
            pl.BlockSpec((64, LBLK), lambda i: (0, i)),
        ],
        out_specs=[
            pl.BlockSpec((128, LBLK), lambda i: (0, i)),
            pl.BlockSpec((128, 1), lambda i: (0, 0)),
            pl.BlockSpec((128, 1), lambda i: (0, 0)),
        ],
        out_shape=[
            jax.ShapeDtypeStruct((128, S), jnp.float32),
            jax.ShapeDtypeStruct((128, 1), jnp.float32),
            jax.ShapeDtypeStruct((128, 1), jnp.float32),
        ],
    )(W2.T, cvec(b2), cvec(g1), cvec(be1), s1, q1, h1)

    mx, mn, s3, q3 = pl.pallas_call(
        functools.partial(_l3_body, n_total=float(S)),
        grid=(_GROUP_SIZE,),
        in_specs=[
            pl.BlockSpec((_HIDDEN, 128), lambda j: (0, 0)),
            pl.BlockSpec((_HIDDEN, 1), lambda j: (0, 0)),
            pl.BlockSpec((128, 1), lambda j: (0, 0)),
            pl.BlockSpec((128, 1), lambda j: (0, 0)),
            pl.BlockSpec((128, 1), lambda j: (0, 0)),
            pl.BlockSpec((128, 1), lambda j: (0, 0)),
            pl.BlockSpec((128, NG), lambda j: (0, j)),
        ],
        out_specs=[
            pl.BlockSpec((_HIDDEN, NG), lambda j: (0, 0)),
            pl.BlockSpec((_HIDDEN, NG), lambda j: (0, 0)),
            pl.BlockSpec((_HIDDEN, 1), lambda j: (0, 0)),
            pl.BlockSpec((_HIDDEN, 1), lambda j: (0, 0)),
        ],
        out_shape=[
            jax.ShapeDtypeStruct((_HIDDEN, NG), jnp.float32),
            jax.ShapeDtypeStruct((_HIDDEN, NG), jnp.float32),
            jax.ShapeDtypeStruct((_HIDDEN, 1), jnp.float32),
            jax.ShapeDtypeStruct((_HIDDEN, 1), jnp.float32),
        ],
    )(W3.T, cvec(b3), cvec(g2), cvec(be2), s2, q2, h2)

    tok_t = pl.pallas_call(
        functools.partial(_fin_body, n_total=float(S)),
        out_shape=jax.ShapeDtypeStruct((_HIDDEN, NG), jnp.float32),
    )(cvec(g3), cvec(be3), s3, q3, mx, mn)
    return tok_t


_NTILE = 32
_CPT = 64
_N = 8192
_NCHUNK = _N // 16


def _knn_sc_body(pts_hbm, cen_hbm, out_hbm,
                 X, Y, Z, CX, CY, CZ, d2i, subhist, histc, sel,
                 cua, cia, cub, cib, outbuf):
    NC = 2
    wid = lax.axis_index("s") * NC + lax.axis_index("c")
    b = wid // 4
    mb = (wid % 4) * _CPT

    pltpu.sync_copy(pts_hbm.at[pl.ds((b * 3 + 0) * _N, _N)], X)
    pltpu.sync_copy(pts_hbm.at[pl.ds((b * 3 + 1) * _N, _N)], Y)
    pltpu.sync_copy(pts_hbm.at[pl.ds((b * 3 + 2) * _N, _N)], Z)
    pltpu.sync_copy(cen_hbm.at[pl.ds((b * 3 + 0) * _NUM_GROUPS + mb, _CPT)],
                    CX.at[pl.ds(0, _CPT)])
    pltpu.sync_copy(cen_hbm.at[pl.ds((b * 3 + 1) * _NUM_GROUPS + mb, _CPT)],
                    CY.at[pl.ds(0, _CPT)])
    pltpu.sync_copy(cen_hbm.at[pl.ds((b * 3 + 2) * _NUM_GROUPS + mb, _CPT)],
                    CZ.at[pl.ds(0, _CPT)])

    iota16 = lax.iota(jnp.int32, 16)
    ones16 = jnp.ones((16,), jnp.int32)
    zeros16 = jnp.zeros((16,), jnp.int32)
    lanebase = iota16 * 256

    def zero_subhist():
        def z(t, c):
            for l in range(16):
                subhist[pl.ds(t * 256 + l * 16, 16)] = zeros16
            return c
        lax.fori_loop(0, 16, z, 0)

    def reduce_hist():
        def red(t, c):
            s = zeros16
            for l in range(16):
                s = s + subhist[pl.ds(l * 256 + t * 16, 16)]
            histc[pl.ds(t * 16, 16)] = s
            return c
        lax.fori_loop(0, 16, red, 0)

    def find_threshold(k_rem):
        def th(t, carry):
            base, T = carry
            h = histc[pl.ds(t * 16, 16)]
            cums = plsc.cumsum(h) + base
            glane = iota16 + t * 16
            candT = jnp.min(jnp.where(cums >= k_rem, glane, 9999))
            return base + jnp.sum(h), jnp.minimum(T, candT)
        _, T = lax.fori_loop(0, 16, th, (jnp.int32(0), jnp.int32(9999)))
        return T

    def row(m, carry0):
        msplat = jnp.full((16,), m, jnp.int32)
        cxv = plsc.load_gather(CX, [msplat])
        cyv = plsc.load_gather(CY, [msplat])
        czv = plsc.load_gather(CZ, [msplat])

        zero_subhist()

        def p1(t8, c):
            for dt in range(8):
                off = (t8 * 8 + dt) * 16
                dx = X[pl.ds(off, 16)] - cxv
                dy = Y[pl.ds(off, 16)] - cyv
                dz = Z[pl.ds(off, 16)] - czv
                d2 = dx * dx + dy * dy + dz * dz
                u = plsc.bitcast(d2, jnp.int32)
                d2i[pl.ds(off, 16)] = u
                bkt = lax.shift_right_logical(u, 23)
                plsc.addupdate_scatter(subhist, [lanebase + bkt], ones16)
            return c
        lax.fori_loop(0, _NCHUNK // 8, p1, 0)

        reduce_hist()
        T1 = find_threshold(jnp.int32(_GROUP_SIZE))

        def cp1(t4, carry):
            selcnt, nout = carry
            for dt in range(4):
                t = t4 * 4 + dt
                u = d2i[pl.ds(t * 16, 16)]
                glane = iota16 + t * 16
                bkt = lax.shift_right_logical(u, 23)
                selm = bkt < T1
                eqm = bkt == T1
                plsc.store_compressed(sel.at[pl.ds(selcnt, 16)], glane,
                                      mask=selm)
                plsc.store_compressed(cua.at[pl.ds(nout, 16)], u, mask=eqm)
                plsc.store_compressed(cia.at[pl.ds(nout, 16)], glane,
                                      mask=eqm)
                selcnt = selcnt + jnp.sum(selm.astype(jnp.int32))
                nout = nout + jnp.sum(eqm.astype(jnp.int32))
            return (selcnt, nout)
        selcnt, ncand = lax.fori_loop(0, _NCHUNK // 4, cp1,
                                      (jnp.int32(0), jnp.int32(0)))

        bufs = [(cua, cia, cub, cib), (cub, cib, cua, cia), (cua, cia, cub, cib)]
        for (inu, ini, outu, outi), shift in zip(bufs, (15, 7, 0)):
            k_rem = _GROUP_SIZE - selcnt
            nch = (ncand + 15) >> 4
            zero_subhist()

            def h2(t, c, inu=inu, shift=shift, ncand=ncand):
                u = inu[pl.ds(t * 16, 16)]
                glane = iota16 + t * 16
                valid = glane < ncand
                bkt = lax.shift_right_logical(u, shift) & 0xFF
                plsc.addupdate_scatter(subhist, [lanebase + bkt], ones16,
                                       mask=valid)
                return c
            lax.fori_loop(0, nch, h2, 0)

            reduce_hist()
            T = find_threshold(k_rem)

            def cp(t, carry, inu=inu, ini=ini, outu=outu, outi=outi,
                   shift=shift, ncand=ncand, T=T):
                scnt, nout = carry
                u = inu[pl.ds(t * 16, 16)]
                ii = ini[pl.ds(t * 16, 16)]
                glane = iota16 + t * 16
                valid = glane < ncand
                bkt = lax.shift_right_logical(u, shift) & 0xFF
                selm = valid & (bkt < T)
                eqm = valid & (bkt == T)
                plsc.store_compressed(sel.at[pl.ds(scnt, 16)], ii, mask=selm)
                plsc.store_compressed(outu.at[pl.ds(nout, 16)], u, mask=eqm)
                plsc.store_compressed(outi.at[pl.ds(nout, 16)], ii, mask=eqm)
                return (scnt + jnp.sum(selm.astype(jnp.int32)),
                        nout + jnp.sum(eqm.astype(jnp.int32)))
            selcnt, ncand = lax.fori_loop(0, nch, cp, (selcnt, jnp.int32(0)))

        k_rem = _GROUP_SIZE - selcnt

        def fill(t, scnt):
            ii = cib[pl.ds(t * 16, 16)]
            glane = iota16 + t * 16
            maskf = glane < k_rem
            plsc.store_compressed(sel.at[pl.ds(scnt, 16)], ii, mask=maskf)
            return scnt + jnp.sum(maskf.astype(jnp.int32))
        selcnt = lax.fori_loop(0, 2, fill, selcnt)

        def gat(c, _):
            idxv = sel[pl.ds(c * 16, 16)]
            jv = iota16 + c * 16
            px = plsc.load_gather(X, [idxv]) - cxv
            py = plsc.load_gather(Y, [idxv]) - cyv
            pz = plsc.load_gather(Z, [idxv]) - czv
            zsp = jnp.zeros((16,), jnp.int32)
            plsc.store_scatter(outbuf, [zsp, jv, msplat], px)
            plsc.store_scatter(outbuf, [zsp + 1, jv, msplat], py)
            plsc.store_scatter(outbuf, [zsp + 2, jv, msplat], pz)
            return _
        lax.fori_loop(0, 2, gat, 0)
        return carry0

    lax.fori_loop(0, _CPT, row, 0)
    pltpu.sync_copy(outbuf, out_hbm.at[wid])


def _knn_sc(points, cen_t):
    B = points.shape[0]
    pts_flat = points.transpose(0, 2, 1).reshape(B * 3 * _N)
    cen_flat = cen_t.transpose(1, 0, 2).reshape(B * 3 * _NUM_GROUPS)
    mesh = plsc.VectorSubcoreMesh(core_axis_name="c", subcore_axis_name="s")
    f32, i32 = jnp.float32, jnp.int32
    out = pl.kernel(
        _knn_sc_body,
        mesh=mesh,
        out_type=jax.ShapeDtypeStruct((_NTILE, 3, _GROUP_SIZE, _CPT), f32),
        compiler_params=pltpu.CompilerParams(needs_layout_passes=False),
        scratch_types=[
            pltpu.VMEM((_N,), f32), pltpu.VMEM((_N,), f32),
            pltpu.VMEM((_N,), f32),
            pltpu.VMEM((128,), f32), pltpu.VMEM((128,), f32),
            pltpu.VMEM((128,), f32),
            pltpu.VMEM((_N,), i32),
            pltpu.VMEM((4096,), i32),
            pltpu.VMEM((256,), i32),
            pltpu.VMEM((48,), i32),
            pltpu.VMEM((_N + 16,), i32), pltpu.VMEM((_N + 16,), i32),
            pltpu.VMEM((_N + 16,), i32), pltpu.VMEM((_N + 16,), i32),
            pltpu.VMEM((3, _GROUP_SIZE, _CPT), f32),
        ],
    )(pts_flat, cen_flat)
    G = out.transpose(1, 2, 0, 3).reshape(3, _GROUP_SIZE * B * _NUM_GROUPS)
    return G


def kernel(points, W1, b1, g1, be1, W2, b2, g2, be2, W3, b3, g3, be3):
    B, N, _ = points.shape
    cen_t = _fps(points)
    centers = cen_t.transpose(1, 2, 0)

    G = _knn_sc(points, cen_t)

    tok_t = _pointnet(G, W1, b1, g1, be1, W2, b2, g2, be2, W3, b3, g3, be3)
    tokens = tok_t.T.reshape(B, _NUM_GROUPS, _HIDDEN)
    return (tokens, centers)


# speedup vs baseline: 1.3573x; 1.2413x over previous
import functools

import jax
import jax.numpy as jnp
from jax import lax
from jax.experimental import pallas as pl
from jax.experimental.pallas import tpu as pltpu
from jax.experimental.pallas import tpu_sc as plsc

_NUM_GROUPS = 256
_GROUP_SIZE = 32
_HIDDEN = 384
_EPS = 1e-5


def _fps_body(pts_ref, f0_ref, cen_ref):
    X = pts_ref[0]
    Y = pts_ref[1]
    Z = pts_ref[2]
    B, N = X.shape
    col = jax.lax.broadcasted_iota(jnp.int32, (B, N), 1)
    colc = jax.lax.broadcasted_iota(jnp.int32, (B, _NUM_GROUPS), 1)
    cen_ref[...] = jnp.zeros_like(cen_ref)

    def body(i, carry):
        dist, f = carry
        onehot = col == f
        cx = jnp.sum(jnp.where(onehot, X, 0.0), axis=1, keepdims=True)
        cy = jnp.sum(jnp.where(onehot, Y, 0.0), axis=1, keepdims=True)
        cz = jnp.sum(jnp.where(onehot, Z, 0.0), axis=1, keepdims=True)
        sel = colc == i
        cen_ref[0] = jnp.where(sel, cx, cen_ref[0])
        cen_ref[1] = jnp.where(sel, cy, cen_ref[1])
        cen_ref[2] = jnp.where(sel, cz, cen_ref[2])
        d = (X - cx) ** 2 + (Y - cy) ** 2 + (Z - cz) ** 2
        dist = jnp.minimum(dist, d)
        m = jnp.max(dist, axis=1, keepdims=True)
        f = jnp.min(jnp.where(dist == m, col, N), axis=1, keepdims=True)
        return dist, f

    dist0 = jnp.full((B, N), 1e10, dtype=jnp.float32)
    f0 = f0_ref[...]
    jax.lax.fori_loop(0, _NUM_GROUPS, body, (dist0, f0))


def _fps(points):
    B, N, _ = points.shape
    pts_t = points.transpose(2, 0, 1)
    f0 = jax.random.randint(jax.random.key(42), (B,), 0, N).astype(jnp.int32)
    cen_t = pl.pallas_call(
        _fps_body,
        out_shape=jax.ShapeDtypeStruct((3, B, _NUM_GROUPS), jnp.float32),
    )(pts_t, f0.reshape(B, 1))
    return cen_t


def _l1_body(w1t_ref, b1_ref, g_ref, h1_ref, ssum_ref, ssq_ref):
    i = pl.program_id(0)
    h = jnp.dot(w1t_ref[...], g_ref[...], preferred_element_type=jnp.float32)
    h = h + b1_ref[...]
    h1_ref[...] = h

    @pl.when(i == 0)
    def _():
        ssum_ref[...] = jnp.zeros_like(ssum_ref)
        ssq_ref[...] = jnp.zeros_like(ssq_ref)

    ssum_ref[...] += jnp.sum(h, axis=1, keepdims=True)
    ssq_ref[...] += jnp.sum(h * h, axis=1, keepdims=True)


def _l2_body(w2t_ref, b2_ref, g1_ref, be1_ref, s1_ref, q1_ref, h1_ref,
             h2_ref, ssum_ref, ssq_ref, *, n_total):
    i = pl.program_id(0)
    mean = s1_ref[...] / n_total
    var = q1_ref[...] / n_total - mean * mean
    s = g1_ref[...] * jax.lax.rsqrt(var + _EPS)
    t = be1_ref[...] - mean * s
    a = jax.nn.relu(s * h1_ref[...] + t)
    h = jnp.dot(w2t_ref[...], a, preferred_element_type=jnp.float32)
    h = h + b2_ref[...]
    h2_ref[...] = h

    @pl.when(i == 0)
    def _():
        ssum_ref[...] = jnp.zeros_like(ssum_ref)
        ssq_ref[...] = jnp.zeros_like(ssq_ref)

    ssum_ref[...] += jnp.sum(h, axis=1, keepdims=True)
    ssq_ref[...] += jnp.sum(h * h, axis=1, keepdims=True)


def _l3_body(w3t_ref, b3_ref, g2_ref, be2_ref, s2_ref, q2_ref, h2_ref,
             mx_ref, mn_ref, ssum_ref, ssq_ref, *, n_total):
    j = pl.program_id(0)
    mean = s2_ref[...] / n_total
    var = q2_ref[...] / n_total - mean * mean
    s = g2_ref[...] * jax.lax.rsqrt(var + _EPS)
    t = be2_ref[...] - mean * s
    a = jax.nn.relu(s * h2_ref[...] + t)
    h = jnp.dot(w3t_ref[...], a, preferred_element_type=jnp.float32)
    h = h + b3_ref[...]

    @pl.when(j == 0)
    def _():
        ssum_ref[...] = jnp.zeros_like(ssum_ref)
        ssq_ref[...] = jnp.zeros_like(ssq_ref)
        mx_ref[...] = jnp.full_like(mx_ref, -jnp.inf)
        mn_ref[...] = jnp.full_like(mn_ref, jnp.inf)

    ssum_ref[...] += jnp.sum(h, axis=1, keepdims=True)
    ssq_ref[...] += jnp.sum(h * h, axis=1, keepdims=True)
    mx_ref[...] = jnp.maximum(mx_ref[...], h)
    mn_ref[...] = jnp.minimum(mn_ref[...], h)


def _fin_body(g3_ref, be3_ref, s3_ref, q3_ref, mx_ref, mn_ref, tok_ref, *, n_total):
    mean = s3_ref[...] / n_total
    var = q3_ref[...] / n_total - mean * mean
    s = g3_ref[...] * jax.lax.rsqrt(var + _EPS)
    t = be3_ref[...] - mean * s
    picked = jnp.where(s >= 0.0, mx_ref[...], mn_ref[...])
    tok_ref[...] = picked * s + t


def _pointnet(G, W1, b1, g1, be1, W2, b2, g2, be2, W3, b3, g3, be3):
    S = G.shape[1]
    NG = S // _GROUP_SIZE
    cvec = lambda v: v.reshape(-1, 1)

    LBLK = 8192
    nblk = S // LBLK
    h1, s1, q1 = pl.pallas_call(
        _l1_body,
        grid=(nblk,),
        in_specs=[
            pl.BlockSpec((64, 3), lambda i: (0, 0)),
            pl.BlockSpec((64, 1), lambda i: (0, 0)),
            pl.BlockSpec((3, LBLK), lambda i: (0, i)),
        ],
        out_specs=[
            pl.BlockSpec((64, LBLK), lambda i: (0, i)),
            pl.BlockSpec((64, 1), lambda i: (0, 0)),
            pl.BlockSpec((64, 1), lambda i: (0, 0)),
        ],
        out_shape=[
            jax.ShapeDtypeStruct((64, S), jnp.float32),
            jax.ShapeDtypeStruct((64, 1), jnp.float32),
            jax.ShapeDtypeStruct((64, 1), jnp.float32),
        ],
    )(W1.T, cvec(b1), G)

    h2, s2, q2 = pl.pallas_call(
        functools.partial(_l2_body, n_total=float(S)),
        grid=(nblk,),
        in_specs=[
            pl.BlockSpec((128, 64), lambda i: (0, 0)),
            pl.BlockSpec((128, 1), lambda i: (0, 0)),
            pl.BlockSpec((64, 1), lambda i: (0, 0)),
            pl.BlockSpec((64, 1), lambda i: (0, 0)),
            pl.BlockSpec((64, 1), lambda i: (0, 0)),
            pl.BlockSpec((64, 1), lambda i: (0, 0)),
            pl.BlockSpec((64, LBLK), lambda i: (0, i)),
        ],
        out_specs=[
            pl.BlockSpec((128, LBLK), lambda i: (0, i)),
            pl.BlockSpec((128, 1), lambda i: (0, 0)),
            pl.BlockSpec((128, 1), lambda i: (0, 0)),
        ],
        out_shape=[
            jax.ShapeDtypeStruct((128, S), jnp.float32),
            jax.ShapeDtypeStruct((128, 1), jnp.float32),
            jax.ShapeDtypeStruct((128, 1), jnp.float32),
        ],
    )(W2.T, cvec(b2), cvec(g1), cvec(be1), s1, q1, h1)

    mx, mn, s3, q3 = pl.pallas_call(
        functools.partial(_l3_body, n_total=float(S)),
        grid=(_GROUP_SIZE,),
        in_specs=[
            pl.BlockSpec((_HIDDEN, 128), lambda j: (0, 0)),
            pl.BlockSpec((_HIDDEN, 1), lambda j: (0, 0)),
            pl.BlockSpec((128, 1), lambda j: (0, 0)),
            pl.BlockSpec((128, 1), lambda j: (0, 0)),
            pl.BlockSpec((128, 1), lambda j: (0, 0)),
            pl.BlockSpec((128, 1), lambda j: (0, 0)),
            pl.BlockSpec((128, NG), lambda j: (0, j)),
        ],
        out_specs=[
            pl.BlockSpec((_HIDDEN, NG), lambda j: (0, 0)),
            pl.BlockSpec((_HIDDEN, NG), lambda j: (0, 0)),
            pl.BlockSpec((_HIDDEN, 1), lambda j: (0, 0)),
            pl.BlockSpec((_HIDDEN, 1), lambda j: (0, 0)),
        ],
        out_shape=[
            jax.ShapeDtypeStruct((_HIDDEN, NG), jnp.float32),
            jax.ShapeDtypeStruct((_HIDDEN, NG), jnp.float32),
            jax.ShapeDtypeStruct((_HIDDEN, 1), jnp.float32),
            jax.ShapeDtypeStruct((_HIDDEN, 1), jnp.float32),
        ],
    )(W3.T, cvec(b3), cvec(g2), cvec(be2), s2, q2, h2)

    tok_t = pl.pallas_call(
        functools.partial(_fin_body, n_total=float(S)),
        out_shape=jax.ShapeDtypeStruct((_HIDDEN, NG), jnp.float32),
    )(cvec(g3), cvec(be3), s3, q3, mx, mn)
    return tok_t


_NTILE = 32
_CPT = 64
_N = 8192
_NCHUNK = _N // 16


def _knn_sc_body(pts_hbm, cen_hbm, out_hbm,
                 X, Y, Z, CX, CY, CZ, d2i, subhist, histc, sel,
                 cmins, qlist, cua, cia, cub, cib, outbuf):
    NC = 2
    wid = lax.axis_index("s") * NC + lax.axis_index("c")
    b = wid // 4
    mb = (wid % 4) * _CPT

    pltpu.sync_copy(pts_hbm.at[pl.ds((b * 3 + 0) * _N, _N)], X)
    pltpu.sync_copy(pts_hbm.at[pl.ds((b * 3 + 1) * _N, _N)], Y)
    pltpu.sync_copy(pts_hbm.at[pl.ds((b * 3 + 2) * _N, _N)], Z)
    pltpu.sync_copy(cen_hbm.at[pl.ds((b * 3 + 0) * _NUM_GROUPS + mb, _CPT)],
                    CX.at[pl.ds(0, _CPT)])
    pltpu.sync_copy(cen_hbm.at[pl.ds((b * 3 + 1) * _NUM_GROUPS + mb, _CPT)],
                    CY.at[pl.ds(0, _CPT)])
    pltpu.sync_copy(cen_hbm.at[pl.ds((b * 3 + 2) * _NUM_GROUPS + mb, _CPT)],
                    CZ.at[pl.ds(0, _CPT)])

    MAXI = jnp.int32(0x7FFFFFFF)
    iota16 = lax.iota(jnp.int32, 16)
    ones16 = jnp.ones((16,), jnp.int32)
    zeros16 = jnp.zeros((16,), jnp.int32)
    lanebase = iota16 * 256

    def zero_subhist():
        def z(t, c):
            for l in range(16):
                subhist[pl.ds(t * 256 + l * 16, 16)] = zeros16
            return c
        lax.fori_loop(0, 16, z, 0)

    def reduce_hist():
        def red(t, c):
            s = zeros16
            for l in range(16):
                s = s + subhist[pl.ds(l * 256 + t * 16, 16)]
            histc[pl.ds(t * 16, 16)] = s
            return c
        lax.fori_loop(0, 16, red, 0)

    def find_threshold(k_rem):
        def th(t, carry):
            base, T = carry
            h = histc[pl.ds(t * 16, 16)]
            cums = plsc.cumsum(h) + base
            glane = iota16 + t * 16
            candT = jnp.min(jnp.where(cums >= k_rem, glane, 9999))
            return base + jnp.sum(h), jnp.minimum(T, candT)
        _, T = lax.fori_loop(0, 16, th, (jnp.int32(0), jnp.int32(9999)))
        return T

    def row(m, carry0):
        msplat = jnp.full((16,), m, jnp.int32)
        cxv = plsc.load_gather(CX, [msplat])
        cyv = plsc.load_gather(CY, [msplat])
        czv = plsc.load_gather(CZ, [msplat])

        zero_subhist()

        def p1(g, c):
            acc = jnp.full((16,), MAXI, jnp.int32)
            for dt in range(16):
                off = (g * 16 + dt) * 16
                dx = X[pl.ds(off, 16)] - cxv
                dy = Y[pl.ds(off, 16)] - cyv
                dz = Z[pl.ds(off, 16)] - czv
                d2 = dx * dx + dy * dy + dz * dz
                u = plsc.bitcast(d2, jnp.int32)
                d2i[pl.ds(off, 16)] = u
                bkt = lax.shift_right_logical(u, 23)
                plsc.addupdate_scatter(subhist, [lanebase + bkt], ones16)
                acc = jnp.where(iota16 == dt, jnp.min(u), acc)
            cmins[pl.ds(g * 16, 16)] = acc
            return c
        lax.fori_loop(0, _NCHUNK // 16, p1, 0)

        reduce_hist()
        T1 = find_threshold(jnp.int32(_GROUP_SIZE))

        qb = jnp.where(T1 >= 255, MAXI, (T1 + 1) << 23)

        def qual(t, nq):
            v = cmins[pl.ds(t * 16, 16)]
            cidv = iota16 + t * 16
            maskq = v < qb
            plsc.store_compressed(qlist.at[pl.ds(nq, 16)], cidv, mask=maskq)
            return nq + jnp.sum(maskq.astype(jnp.int32))
        nq = lax.fori_loop(0, _NCHUNK // 16, qual, jnp.int32(0))

        def cq(t, carry):
            selcnt, nout = carry
            tsp = jnp.full((16,), t, jnp.int32)
            cid = jnp.max(plsc.load_gather(qlist, [tsp]))
            u = d2i[pl.ds(cid * 16, 16)]
            glane = iota16 + cid * 16
            bkt = lax.shift_right_logical(u, 23)
            selm = bkt < T1
            eqm = bkt == T1
            plsc.store_compressed(sel.at[pl.ds(selcnt, 16)], glane, mask=selm)
            plsc.store_compressed(cua.at[pl.ds(nout, 16)], u, mask=eqm)
            plsc.store_compressed(cia.at[pl.ds(nout, 16)], glane, mask=eqm)
            return (jnp.minimum(selcnt + jnp.sum(selm.astype(jnp.int32)),
                                jnp.int32(32)),
                    jnp.minimum(nout + jnp.sum(eqm.astype(jnp.int32)),
                                jnp.int32(_N)))
        selcnt, ncand = lax.fori_loop(0, nq, cq,
                                      (jnp.int32(0), jnp.int32(0)))

        bufs = [(cua, cia, cub, cib), (cub, cib, cua, cia), (cua, cia, cub, cib)]
        for (inu, ini, outu, outi), shift in zip(bufs, (15, 7, 0)):
            k_rem = _GROUP_SIZE - selcnt
            nch = (ncand + 15) >> 4
            zero_subhist()

            def h2(t, c, inu=inu, shift=shift, ncand=ncand):
                u = inu[pl.ds(t * 16, 16)]
                glane = iota16 + t * 16
                valid = glane < ncand
                bkt = lax.shift_right_logical(u, shift) & 0xFF
                plsc.addupdate_scatter(subhist, [lanebase + bkt], ones16,
                                       mask=valid)
                return c
            lax.fori_loop(0, nch, h2, 0)

            reduce_hist()
            T = find_threshold(k_rem)

            def cp(t, carry, inu=inu, ini=ini, outu=outu, outi=outi,
                   shift=shift, ncand=ncand, T=T):
                scnt, nout = carry
                u = inu[pl.ds(t * 16, 16)]
                ii = ini[pl.ds(t * 16, 16)]
                glane = iota16 + t * 16
                valid = glane < ncand
                bkt = lax.shift_right_logical(u, shift) & 0xFF
                selm = valid & (bkt < T)
                eqm = valid & (bkt == T)
                plsc.store_compressed(sel.at[pl.ds(scnt, 16)], ii, mask=selm)
                plsc.store_compressed(outu.at[pl.ds(nout, 16)], u, mask=eqm)
                plsc.store_compressed(outi.at[pl.ds(nout, 16)], ii, mask=eqm)
                return (jnp.minimum(scnt + jnp.sum(selm.astype(jnp.int32)),
                                    jnp.int32(32)),
                        jnp.minimum(nout + jnp.sum(eqm.astype(jnp.int32)),
                                    jnp.int32(_N)))
            selcnt, ncand = lax.fori_loop(0, nch, cp, (selcnt, jnp.int32(0)))

        k_rem = _GROUP_SIZE - selcnt

        def fill(t, scnt):
            ii = cib[pl.ds(t * 16, 16)]
            glane = iota16 + t * 16
            maskf = glane < k_rem
            plsc.store_compressed(sel.at[pl.ds(scnt, 16)], ii, mask=maskf)
            return scnt + jnp.sum(maskf.astype(jnp.int32))
        selcnt = lax.fori_loop(0, 2, fill, selcnt)

        def gat(c, _):
            idxv = sel[pl.ds(c * 16, 16)]
            jv = iota16 + c * 16
            px = plsc.load_gather(X, [idxv]) - cxv
            py = plsc.load_gather(Y, [idxv]) - cyv
            pz = plsc.load_gather(Z, [idxv]) - czv
            zsp = jnp.zeros((16,), jnp.int32)
            plsc.store_scatter(outbuf, [zsp, jv, msplat], px)
            plsc.store_scatter(outbuf, [zsp + 1, jv, msplat], py)
            plsc.store_scatter(outbuf, [zsp + 2, jv, msplat], pz)
            return _
        lax.fori_loop(0, 2, gat, 0)
        return carry0

    lax.fori_loop(0, _CPT, row, 0)
    pltpu.sync_copy(outbuf, out_hbm.at[wid])


def _knn_sc(points, cen_t):
    B = points.shape[0]
    pts_flat = points.transpose(0, 2, 1).reshape(B * 3 * _N)
    cen_flat = cen_t.transpose(1, 0, 2).reshape(B * 3 * _NUM_GROUPS)
    mesh = plsc.VectorSubcoreMesh(core_axis_name="c", subcore_axis_name="s")
    f32, i32 = jnp.float32, jnp.int32
    out = pl.kernel(
        _knn_sc_body,
        mesh=mesh,
        out_type=jax.ShapeDtypeStruct((_NTILE, 3, _GROUP_SIZE, _CPT), f32),
        compiler_params=pltpu.CompilerParams(needs_layout_passes=False),
        scratch_types=[
            pltpu.VMEM((_N,), f32), pltpu.VMEM((_N,), f32),
            pltpu.VMEM((_N,), f32),
            pltpu.VMEM((128,), f32), pltpu.VMEM((128,), f32),
            pltpu.VMEM((128,), f32),
            pltpu.VMEM((_N,), i32),
            pltpu.VMEM((4096,), i32),
            pltpu.VMEM((256,), i32),
            pltpu.VMEM((48,), i32),
            pltpu.VMEM((_NCHUNK,), i32),
            pltpu.VMEM((_NCHUNK + 16,), i32),
            pltpu.VMEM((_N + 16,), i32), pltpu.VMEM((_N + 16,), i32),
            pltpu.VMEM((_N + 16,), i32), pltpu.VMEM((_N + 16,), i32),
            pltpu.VMEM((3, _GROUP_SIZE, _CPT), f32),
        ],
    )(pts_flat, cen_flat)
    G = out.transpose(1, 2, 0, 3).reshape(3, _GROUP_SIZE * B * _NUM_GROUPS)
    return G


def kernel(points, W1, b1, g1, be1, W2, b2, g2, be2, W3, b3, g3, be3):
    B, N, _ = points.shape
    cen_t = _fps(points)
    centers = cen_t.transpose(1, 2, 0)

    G = _knn_sc(points, cen_t)

    tok_t = _pointnet(G, W1, b1, g1, be1, W2, b2, g2, be2, W3, b3, g3, be3)
    tokens = tok_t.T.reshape(B, _NUM_GROUPS, _HIDDEN)
    return (tokens, centers)


# speedup vs baseline: 1.9056x; 1.4040x over previous
import functools

import jax
import jax.numpy as jnp
from jax import lax
from jax.experimental import pallas as pl
from jax.experimental.pallas import tpu as pltpu
from jax.experimental.pallas import tpu_sc as plsc

_NUM_GROUPS = 256
_GROUP_SIZE = 32
_HIDDEN = 384
_EPS = 1e-5


def _fps_body(pts_ref, f0_ref, cen_ref):
    X = pts_ref[0]
    Y = pts_ref[1]
    Z = pts_ref[2]
    B, N = X.shape
    col = jax.lax.broadcasted_iota(jnp.int32, (B, N), 1)
    colc = jax.lax.broadcasted_iota(jnp.int32, (B, _NUM_GROUPS), 1)
    cen_ref[...] = jnp.zeros_like(cen_ref)

    def body(i, carry):
        dist, f = carry
        onehot = col == f
        cx = jnp.sum(jnp.where(onehot, X, 0.0), axis=1, keepdims=True)
        cy = jnp.sum(jnp.where(onehot, Y, 0.0), axis=1, keepdims=True)
        cz = jnp.sum(jnp.where(onehot, Z, 0.0), axis=1, keepdims=True)
        sel = colc == i
        cen_ref[0] = jnp.where(sel, cx, cen_ref[0])
        cen_ref[1] = jnp.where(sel, cy, cen_ref[1])
        cen_ref[2] = jnp.where(sel, cz, cen_ref[2])
        d = (X - cx) ** 2 + (Y - cy) ** 2 + (Z - cz) ** 2
        dist = jnp.minimum(dist, d)
        m = jnp.max(dist, axis=1, keepdims=True)
        f = jnp.min(jnp.where(dist == m, col, N), axis=1, keepdims=True)
        return dist, f

    dist0 = jnp.full((B, N), 1e10, dtype=jnp.float32)
    f0 = f0_ref[...]
    jax.lax.fori_loop(0, _NUM_GROUPS, body, (dist0, f0))


def _fps(points):
    B, N, _ = points.shape
    pts_t = points.transpose(2, 0, 1)
    f0 = jax.random.randint(jax.random.key(42), (B,), 0, N).astype(jnp.int32)
    cen_t = pl.pallas_call(
        _fps_body,
        out_shape=jax.ShapeDtypeStruct((3, B, _NUM_GROUPS), jnp.float32),
    )(pts_t, f0.reshape(B, 1))
    return cen_t


def _l1_body(w1t_ref, b1_ref, g_ref, h1_ref, ssum_ref, ssq_ref):
    i = pl.program_id(0)
    h = jnp.dot(w1t_ref[...], g_ref[...], preferred_element_type=jnp.float32)
    h = h + b1_ref[...]
    h1_ref[...] = h

    @pl.when(i == 0)
    def _():
        ssum_ref[...] = jnp.zeros_like(ssum_ref)
        ssq_ref[...] = jnp.zeros_like(ssq_ref)

    ssum_ref[...] += jnp.sum(h, axis=1, keepdims=True)
    ssq_ref[...] += jnp.sum(h * h, axis=1, keepdims=True)


def _l2_body(w2t_ref, b2_ref, g1_ref, be1_ref, s1_ref, q1_ref, h1_ref,
             h2_ref, ssum_ref, ssq_ref, *, n_total):
    i = pl.program_id(0)
    mean = s1_ref[...] / n_total
    var = q1_ref[...] / n_total - mean * mean
    s = g1_ref[...] * jax.lax.rsqrt(var + _EPS)
    t = be1_ref[...] - mean * s
    a = jax.nn.relu(s * h1_ref[...] + t)
    h = jnp.dot(w2t_ref[...], a, preferred_element_type=jnp.float32)
    h = h + b2_ref[...]
    h2_ref[...] = h

    @pl.when(i == 0)
    def _():
        ssum_ref[...] = jnp.zeros_like(ssum_ref)
        ssq_ref[...] = jnp.zeros_like(ssq_ref)

    ssum_ref[...] += jnp.sum(h, axis=1, keepdims=True)
    ssq_ref[...] += jnp.sum(h * h, axis=1, keepdims=True)


def _l3_body(w3t_ref, b3_ref, g2_ref, be2_ref, s2_ref, q2_ref, h2_ref,
             mx_ref, mn_ref, ssum_ref, ssq_ref, *, n_total):
    j = pl.program_id(0)
    mean = s2_ref[...] / n_total
    var = q2_ref[...] / n_total - mean * mean
    s = g2_ref[...] * jax.lax.rsqrt(var + _EPS)
    t = be2_ref[...] - mean * s
    a = jax.nn.relu(s * h2_ref[...] + t)
    h = jnp.dot(w3t_ref[...], a, preferred_element_type=jnp.float32)
    h = h + b3_ref[...]

    @pl.when(j == 0)
    def _():
        ssum_ref[...] = jnp.zeros_like(ssum_ref)
        ssq_ref[...] = jnp.zeros_like(ssq_ref)
        mx_ref[...] = jnp.full_like(mx_ref, -jnp.inf)
        mn_ref[...] = jnp.full_like(mn_ref, jnp.inf)

    ssum_ref[...] += jnp.sum(h, axis=1, keepdims=True)
    ssq_ref[...] += jnp.sum(h * h, axis=1, keepdims=True)
    mx_ref[...] = jnp.maximum(mx_ref[...], h)
    mn_ref[...] = jnp.minimum(mn_ref[...], h)


def _fin_body(g3_ref, be3_ref, s3_ref, q3_ref, mx_ref, mn_ref, tok_ref, *, n_total):
    mean = s3_ref[...] / n_total
    var = q3_ref[...] / n_total - mean * mean
    s = g3_ref[...] * jax.lax.rsqrt(var + _EPS)
    t = be3_ref[...] - mean * s
    picked = jnp.where(s >= 0.0, mx_ref[...], mn_ref[...])
    tok_ref[...] = picked * s + t


def _pointnet(G, W1, b1, g1, be1, W2, b2, g2, be2, W3, b3, g3, be3):
    S = G.shape[1]
    NG = S // _GROUP_SIZE
    cvec = lambda v: v.reshape(-1, 1)

    LBLK = 8192
    nblk = S // LBLK
    h1, s1, q1 = pl.pallas_call(
        _l1_body,
        grid=(nblk,),
        in_specs=[
            pl.BlockSpec((64, 3), lambda i: (0, 0)),
            pl.BlockSpec((64, 1), lambda i: (0, 0)),
            pl.BlockSpec((3, LBLK), lambda i: (0, i)),
        ],
        out_specs=[
            pl.BlockSpec((64, LBLK), lambda i: (0, i)),
            pl.BlockSpec((64, 1), lambda i: (0, 0)),
            pl.BlockSpec((64, 1), lambda i: (0, 0)),
        ],
        out_shape=[
            jax.ShapeDtypeStruct((64, S), jnp.float32),
            jax.ShapeDtypeStruct((64, 1), jnp.float32),
            jax.ShapeDtypeStruct((64, 1), jnp.float32),
        ],
    )(W1.T, cvec(b1), G)

    h2, s2, q2 = pl.pallas_call(
        functools.partial(_l2_body, n_total=float(S)),
        grid=(nblk,),
        in_specs=[
            pl.BlockSpec((128, 64), lambda i: (0, 0)),
            pl.BlockSpec((128, 1), lambda i: (0, 0)),
            pl.BlockSpec((64, 1), lambda i: (0, 0)),
            pl.BlockSpec((64, 1), lambda i: (0, 0)),
            pl.BlockSpec((64, 1), lambda i: (0, 0)),
            pl.BlockSpec((64, 1), lambda i: (0, 0)),
            pl.BlockSpec((64, LBLK), lambda i: (0, i)),
        ],
        out_specs=[
            pl.BlockSpec((128, LBLK), lambda i: (0, i)),
            pl.BlockSpec((128, 1), lambda i: (0, 0)),
            pl.BlockSpec((128, 1), lambda i: (0, 0)),
        ],
        out_shape=[
            jax.ShapeDtypeStruct((128, S), jnp.float32),
            jax.ShapeDtypeStruct((128, 1), jnp.float32),
            jax.ShapeDtypeStruct((128, 1), jnp.float32),
        ],
    )(W2.T, cvec(b2), cvec(g1), cvec(be1), s1, q1, h1)

    mx, mn, s3, q3 = pl.pallas_call(
        functools.partial(_l3_body, n_total=float(S)),
        grid=(_GROUP_SIZE,),
        in_specs=[
            pl.BlockSpec((_HIDDEN, 128), lambda j: (0, 0)),
            pl.BlockSpec((_HIDDEN, 1), lambda j: (0, 0)),
            pl.BlockSpec((128, 1), lambda j: (0, 0)),
            pl.BlockSpec((128, 1), lambda j: (0, 0)),
            pl.BlockSpec((128, 1), lambda j: (0, 0)),
            pl.BlockSpec((128, 1), lambda j: (0, 0)),
            pl.BlockSpec((128, NG), lambda j: (0, j)),
        ],
        out_specs=[
            pl.BlockSpec((_HIDDEN, NG), lambda j: (0, 0)),
            pl.BlockSpec((_HIDDEN, NG), lambda j: (0, 0)),
            pl.BlockSpec((_HIDDEN, 1), lambda j: (0, 0)),
            pl.BlockSpec((_HIDDEN, 1), lambda j: (0, 0)),
        ],
        out_shape=[
            jax.ShapeDtypeStruct((_HIDDEN, NG), jnp.float32),
            jax.ShapeDtypeStruct((_HIDDEN, NG), jnp.float32),
            jax.ShapeDtypeStruct((_HIDDEN, 1), jnp.float32),
            jax.ShapeDtypeStruct((_HIDDEN, 1), jnp.float32),
        ],
    )(W3.T, cvec(b3), cvec(g2), cvec(be2), s2, q2, h2)

    tok_t = pl.pallas_call(
        functools.partial(_fin_body, n_total=float(S)),
        out_shape=jax.ShapeDtypeStruct((_HIDDEN, NG), jnp.float32),
    )(cvec(g3), cvec(be3), s3, q3, mx, mn)
    return tok_t


_NTILE = 32
_CPT = 64
_N = 8192
_NCHUNK = _N // 16


def _knn_sc_body(pts_hbm, cen_hbm, out_hbm,
                 X, Y, Z, CX, CY, CZ, d2i, subhist, histc, sel,
                 cmins, qlist, cua, cia, cub, cib, outbuf):
    NC = 2
    wid = lax.axis_index("s") * NC + lax.axis_index("c")
    b = wid // 4
    mb = (wid % 4) * _CPT

    pltpu.sync_copy(pts_hbm.at[pl.ds((b * 3 + 0) * _N, _N)], X)
    pltpu.sync_copy(pts_hbm.at[pl.ds((b * 3 + 1) * _N, _N)], Y)
    pltpu.sync_copy(pts_hbm.at[pl.ds((b * 3 + 2) * _N, _N)], Z)
    pltpu.sync_copy(cen_hbm.at[pl.ds((b * 3 + 0) * _NUM_GROUPS + mb, _CPT)],
                    CX.at[pl.ds(0, _CPT)])
    pltpu.sync_copy(cen_hbm.at[pl.ds((b * 3 + 1) * _NUM_GROUPS + mb, _CPT)],
                    CY.at[pl.ds(0, _CPT)])
    pltpu.sync_copy(cen_hbm.at[pl.ds((b * 3 + 2) * _NUM_GROUPS + mb, _CPT)],
                    CZ.at[pl.ds(0, _CPT)])

    MAXI = jnp.int32(0x7FFFFFFF)
    iota16 = lax.iota(jnp.int32, 16)
    ones16 = jnp.ones((16,), jnp.int32)
    zeros16 = jnp.zeros((16,), jnp.int32)
    lanebase = iota16 * 256

    def zero_subhist():
        @plsc.parallel_loop(0, 16)
        def _z(t):
            for l in range(16):
                subhist[pl.ds(t * 256 + l * 16, 16)] = zeros16

    def reduce_hist():
        @plsc.parallel_loop(0, 16)
        def _red(t):
            s = zeros16
            for l in range(16):
                s = s + subhist[pl.ds(l * 256 + t * 16, 16)]
            histc[pl.ds(t * 16, 16)] = s

    def find_threshold(k_rem):
        def th(t, carry):
            base, T = carry
            h = histc[pl.ds(t * 16, 16)]
            cums = plsc.cumsum(h) + base
            glane = iota16 + t * 16
            candT = jnp.min(jnp.where(cums >= k_rem, glane, 9999))
            return base + jnp.sum(h), jnp.minimum(T, candT)
        _, T = lax.fori_loop(0, 16, th, (jnp.int32(0), jnp.int32(9999)))
        return T

    def row(m, carry0):
        msplat = jnp.full((16,), m, jnp.int32)
        cxv = plsc.load_gather(CX, [msplat])
        cyv = plsc.load_gather(CY, [msplat])
        czv = plsc.load_gather(CZ, [msplat])

        zero_subhist()

        @plsc.parallel_loop(0, _NCHUNK // 16)
        def _p1(g):
            acc = jnp.full((16,), MAXI, jnp.int32)
            for dt in range(16):
                off = (g * 16 + dt) * 16
                dx = X[pl.ds(off, 16)] - cxv
                dy = Y[pl.ds(off, 16)] - cyv
                dz = Z[pl.ds(off, 16)] - czv
                d2 = dx * dx + dy * dy + dz * dz
                u = plsc.bitcast(d2, jnp.int32)
                d2i[pl.ds(off, 16)] = u
                bkt = lax.shift_right_logical(u, 23)
                plsc.addupdate_scatter(subhist, [lanebase + bkt], ones16)
                acc = jnp.where(iota16 == dt, jnp.min(u), acc)
            cmins[pl.ds(g * 16, 16)] = acc

        reduce_hist()
        T1 = find_threshold(jnp.int32(_GROUP_SIZE))

        qb = jnp.where(T1 >= 255, MAXI, (T1 + 1) << 23)

        def qual(t, nq):
            v = cmins[pl.ds(t * 16, 16)]
            cidv = iota16 + t * 16
            maskq = v < qb
            plsc.store_compressed(qlist.at[pl.ds(nq, 16)], cidv, mask=maskq)
            return nq + jnp.sum(maskq.astype(jnp.int32))
        nq = lax.fori_loop(0, _NCHUNK // 16, qual, jnp.int32(0))

        def cq(t, carry):
            selcnt, nout = carry
            tsp = jnp.full((16,), t, jnp.int32)
            cid = jnp.max(plsc.load_gather(qlist, [tsp]))
            u = d2i[pl.ds(cid * 16, 16)]
            glane = iota16 + cid * 16
            bkt = lax.shift_right_logical(u, 23)
            selm = bkt < T1
            eqm = bkt == T1
            plsc.store_compressed(sel.at[pl.ds(selcnt, 16)], glane, mask=selm)
            plsc.store_compressed(cua.at[pl.ds(nout, 16)], u, mask=eqm)
            plsc.store_compressed(cia.at[pl.ds(nout, 16)], glane, mask=eqm)
            return (jnp.minimum(selcnt + jnp.sum(selm.astype(jnp.int32)),
                                jnp.int32(32)),
                    jnp.minimum(nout + jnp.sum(eqm.astype(jnp.int32)),
                                jnp.int32(_N)))
        selcnt, ncand = lax.fori_loop(0, nq, cq,
                                      (jnp.int32(0), jnp.int32(0)))

        bufs = [(cua, cia, cub, cib), (cub, cib, cua, cia), (cua, cia, cub, cib)]
        for (inu, ini, outu, outi), shift in zip(bufs, (15, 7, 0)):
            k_rem = _GROUP_SIZE - selcnt
            nch = (ncand + 15) >> 4
            zero_subhist()

            def h2(t, c, inu=inu, shift=shift, ncand=ncand):
                u = inu[pl.ds(t * 16, 16)]
                glane = iota16 + t * 16
                valid = glane < ncand
                bkt = lax.shift_right_logical(u, shift) & 0xFF
                plsc.addupdate_scatter(subhist, [lanebase + bkt], ones16,
                                       mask=valid)
                return c
            lax.fori_loop(0, nch, h2, 0)

            reduce_hist()
            T = find_threshold(k_rem)

            def cp(t, carry, inu=inu, ini=ini, outu=outu, outi=outi,
                   shift=shift, ncand=ncand, T=T):
                scnt, nout = carry
                u = inu[pl.ds(t * 16, 16)]
                ii = ini[pl.ds(t * 16, 16)]
                glane = iota16 + t * 16
                valid = glane < ncand
                bkt = lax.shift_right_logical(u, shift) & 0xFF
                selm = valid & (bkt < T)
                eqm = valid & (bkt == T)
                plsc.store_compressed(sel.at[pl.ds(scnt, 16)], ii, mask=selm)
                plsc.store_compressed(outu.at[pl.ds(nout, 16)], u, mask=eqm)
                plsc.store_compressed(outi.at[pl.ds(nout, 16)], ii, mask=eqm)
                return (jnp.minimum(scnt + jnp.sum(selm.astype(jnp.int32)),
                                    jnp.int32(32)),
                        jnp.minimum(nout + jnp.sum(eqm.astype(jnp.int32)),
                                    jnp.int32(_N)))
            selcnt, ncand = lax.fori_loop(0, nch, cp, (selcnt, jnp.int32(0)))

        k_rem = _GROUP_SIZE - selcnt

        def fill(t, scnt):
            ii = cib[pl.ds(t * 16, 16)]
            glane = iota16 + t * 16
            maskf = glane < k_rem
            plsc.store_compressed(sel.at[pl.ds(scnt, 16)], ii, mask=maskf)
            return scnt + jnp.sum(maskf.astype(jnp.int32))
        selcnt = lax.fori_loop(0, 2, fill, selcnt)

        def gat(c, _):
            idxv = sel[pl.ds(c * 16, 16)]
            jv = iota16 + c * 16
            px = plsc.load_gather(X, [idxv]) - cxv
            py = plsc.load_gather(Y, [idxv]) - cyv
            pz = plsc.load_gather(Z, [idxv]) - czv
            zsp = jnp.zeros((16,), jnp.int32)
            plsc.store_scatter(outbuf, [zsp, jv, msplat], px)
            plsc.store_scatter(outbuf, [zsp + 1, jv, msplat], py)
            plsc.store_scatter(outbuf, [zsp + 2, jv, msplat], pz)
            return _
        lax.fori_loop(0, 2, gat, 0)
        return carry0

    lax.fori_loop(0, _CPT, row, 0)
    pltpu.sync_copy(outbuf, out_hbm.at[wid])


def _knn_sc(points, cen_t):
    B = points.shape[0]
    pts_flat = points.transpose(0, 2, 1).reshape(B * 3 * _N)
    cen_flat = cen_t.transpose(1, 0, 2).reshape(B * 3 * _NUM_GROUPS)
    mesh = plsc.VectorSubcoreMesh(core_axis_name="c", subcore_axis_name="s")
    f32, i32 = jnp.float32, jnp.int32
    out = pl.kernel(
        _knn_sc_body,
        mesh=mesh,
        out_type=jax.ShapeDtypeStruct((_NTILE, 3, _GROUP_SIZE, _CPT), f32),
        compiler_params=pltpu.CompilerParams(needs_layout_passes=False),
        scratch_types=[
            pltpu.VMEM((_N,), f32), pltpu.VMEM((_N,), f32),
            pltpu.VMEM((_N,), f32),
            pltpu.VMEM((128,), f32), pltpu.VMEM((128,), f32),
            pltpu.VMEM((128,), f32),
            pltpu.VMEM((_N,), i32),
            pltpu.VMEM((4096,), i32),
            pltpu.VMEM((256,), i32),
            pltpu.VMEM((48,), i32),
            pltpu.VMEM((_NCHUNK,), i32),
            pltpu.VMEM((_NCHUNK + 16,), i32),
            pltpu.VMEM((_N + 16,), i32), pltpu.VMEM((_N + 16,), i32),
            pltpu.VMEM((_N + 16,), i32), pltpu.VMEM((_N + 16,), i32),
            pltpu.VMEM((3, _GROUP_SIZE, _CPT), f32),
        ],
    )(pts_flat, cen_flat)
    G = out.transpose(1, 2, 0, 3).reshape(3, _GROUP_SIZE * B * _NUM_GROUPS)
    return G


def kernel(points, W1, b1, g1, be1, W2, b2, g2, be2, W3, b3, g3, be3):
    B, N, _ = points.shape
    cen_t = _fps(points)
    centers = cen_t.transpose(1, 2, 0)

    G = _knn_sc(points, cen_t)

    tok_t = _pointnet(G, W1, b1, g1, be1, W2, b2, g2, be2, W3, b3, g3, be3)
    tokens = tok_t.T.reshape(B, _NUM_GROUPS, _HIDDEN)
    return (tokens, centers)


# speedup vs baseline: 2.0263x; 1.0633x over previous
import functools

import jax
import jax.numpy as jnp
from jax import lax
from jax.experimental import pallas as pl
from jax.experimental.pallas import tpu as pltpu
from jax.experimental.pallas import tpu_sc as plsc

_NUM_GROUPS = 256
_GROUP_SIZE = 32
_HIDDEN = 384
_EPS = 1e-5


def _fps_body(pts_ref, f0_ref, cen_ref):
    X = pts_ref[0]
    Y = pts_ref[1]
    Z = pts_ref[2]
    B, N = X.shape
    col = jax.lax.broadcasted_iota(jnp.int32, (B, N), 1)
    colc = jax.lax.broadcasted_iota(jnp.int32, (B, _NUM_GROUPS), 1)
    cen_ref[...] = jnp.zeros_like(cen_ref)

    def body(i, carry):
        dist, f = carry
        onehot = col == f
        cx = jnp.sum(jnp.where(onehot, X, 0.0), axis=1, keepdims=True)
        cy = jnp.sum(jnp.where(onehot, Y, 0.0), axis=1, keepdims=True)
        cz = jnp.sum(jnp.where(onehot, Z, 0.0), axis=1, keepdims=True)
        sel = colc == i
        cen_ref[0] = jnp.where(sel, cx, cen_ref[0])
        cen_ref[1] = jnp.where(sel, cy, cen_ref[1])
        cen_ref[2] = jnp.where(sel, cz, cen_ref[2])
        d = (X - cx) ** 2 + (Y - cy) ** 2 + (Z - cz) ** 2
        dist = jnp.minimum(dist, d)
        m = jnp.max(dist, axis=1, keepdims=True)
        f = jnp.min(jnp.where(dist == m, col, N), axis=1, keepdims=True)
        return dist, f

    dist0 = jnp.full((B, N), 1e10, dtype=jnp.float32)
    f0 = f0_ref[...]
    jax.lax.fori_loop(0, _NUM_GROUPS, body, (dist0, f0))


def _fps(points):
    B, N, _ = points.shape
    pts_t = points.transpose(2, 0, 1)
    f0 = jax.random.randint(jax.random.key(42), (B,), 0, N).astype(jnp.int32)
    cen_t = pl.pallas_call(
        _fps_body,
        out_shape=jax.ShapeDtypeStruct((3, B, _NUM_GROUPS), jnp.float32),
    )(pts_t, f0.reshape(B, 1))
    return cen_t


def _l1_body(w1t_ref, b1_ref, g_ref, h1_ref, ssum_ref, ssq_ref):
    i = pl.program_id(0)
    h = jnp.dot(w1t_ref[...], g_ref[...], preferred_element_type=jnp.float32)
    h = h + b1_ref[...]
    h1_ref[...] = h

    @pl.when(i == 0)
    def _():
        ssum_ref[...] = jnp.zeros_like(ssum_ref)
        ssq_ref[...] = jnp.zeros_like(ssq_ref)

    ssum_ref[...] += jnp.sum(h, axis=1, keepdims=True)
    ssq_ref[...] += jnp.sum(h * h, axis=1, keepdims=True)


def _l2_body(w2t_ref, b2_ref, g1_ref, be1_ref, s1_ref, q1_ref, h1_ref,
             h2_ref, ssum_ref, ssq_ref, *, n_total):
    i = pl.program_id(0)
    mean = s1_ref[...] / n_total
    var = q1_ref[...] / n_total - mean * mean
    s = g1_ref[...] * jax.lax.rsqrt(var + _EPS)
    t = be1_ref[...] - mean * s
    a = jax.nn.relu(s * h1_ref[...] + t)
    h = jnp.dot(w2t_ref[...], a, preferred_element_type=jnp.float32)
    h = h + b2_ref[...]
    h2_ref[...] = h

    @pl.when(i == 0)
    def _():
        ssum_ref[...] = jnp.zeros_like(ssum_ref)
        ssq_ref[...] = jnp.zeros_like(ssq_ref)

    ssum_ref[...] += jnp.sum(h, axis=1, keepdims=True)
    ssq_ref[...] += jnp.sum(h * h, axis=1, keepdims=True)


def _l3_body(w3t_ref, b3_ref, g2_ref, be2_ref, s2_ref, q2_ref, h2_ref,
             mx_ref, mn_ref, ssum_ref, ssq_ref, *, n_total):
    j = pl.program_id(0)
    mean = s2_ref[...] / n_total
    var = q2_ref[...] / n_total - mean * mean
    s = g2_ref[...] * jax.lax.rsqrt(var + _EPS)
    t = be2_ref[...] - mean * s
    a = jax.nn.relu(s * h2_ref[...] + t)
    h = jnp.dot(w3t_ref[...], a, preferred_element_type=jnp.float32)
    h = h + b3_ref[...]

    @pl.when(j == 0)
    def _():
        ssum_ref[...] = jnp.zeros_like(ssum_ref)
        ssq_ref[...] = jnp.zeros_like(ssq_ref)
        mx_ref[...] = jnp.full_like(mx_ref, -jnp.inf)
        mn_ref[...] = jnp.full_like(mn_ref, jnp.inf)

    ssum_ref[...] += jnp.sum(h, axis=1, keepdims=True)
    ssq_ref[...] += jnp.sum(h * h, axis=1, keepdims=True)
    mx_ref[...] = jnp.maximum(mx_ref[...], h)
    mn_ref[...] = jnp.minimum(mn_ref[...], h)


def _fin_body(g3_ref, be3_ref, s3_ref, q3_ref, mx_ref, mn_ref, tok_ref, *, n_total):
    mean = s3_ref[...] / n_total
    var = q3_ref[...] / n_total - mean * mean
    s = g3_ref[...] * jax.lax.rsqrt(var + _EPS)
    t = be3_ref[...] - mean * s
    picked = jnp.where(s >= 0.0, mx_ref[...], mn_ref[...])
    tok_ref[...] = picked * s + t


def _pointnet(G, W1, b1, g1, be1, W2, b2, g2, be2, W3, b3, g3, be3):
    S = G.shape[1]
    NG = S // _GROUP_SIZE
    cvec = lambda v: v.reshape(-1, 1)

    LBLK = 8192
    nblk = S // LBLK
    h1, s1, q1 = pl.pallas_call(
        _l1_body,
        grid=(nblk,),
        in_specs=[
            pl.BlockSpec((64, 3), lambda i: (0, 0)),
            pl.BlockSpec((64, 1), lambda i: (0, 0)),
            pl.BlockSpec((3, LBLK), lambda i: (0, i)),
        ],
        out_specs=[
            pl.BlockSpec((64, LBLK), lambda i: (0, i)),
            pl.BlockSpec((64, 1), lambda i: (0, 0)),
            pl.BlockSpec((64, 1), lambda i: (0, 0)),
        ],
        out_shape=[
            jax.ShapeDtypeStruct((64, S), jnp.float32),
            jax.ShapeDtypeStruct((64, 1), jnp.float32),
            jax.ShapeDtypeStruct((64, 1), jnp.float32),
        ],
    )(W1.T, cvec(b1), G)

    h2, s2, q2 = pl.pallas_call(
        functools.partial(_l2_body, n_total=float(S)),
        grid=(nblk,),
        in_specs=[
            pl.BlockSpec((128, 64), lambda i: (0, 0)),
            pl.BlockSpec((128, 1), lambda i: (0, 0)),
            pl.BlockSpec((64, 1), lambda i: (0, 0)),
            pl.BlockSpec((64, 1), lambda i: (0, 0)),
            pl.BlockSpec((64, 1), lambda i: (0, 0)),
            pl.BlockSpec((64, 1), lambda i: (0, 0)),
            pl.BlockSpec((64, LBLK), lambda i: (0, i)),
        ],
        out_specs=[
            pl.BlockSpec((128, LBLK), lambda i: (0, i)),
            pl.BlockSpec((128, 1), lambda i: (0, 0)),
            pl.BlockSpec((128, 1), lambda i: (0, 0)),
        ],
        out_shape=[
            jax.ShapeDtypeStruct((128, S), jnp.float32),
            jax.ShapeDtypeStruct((128, 1), jnp.float32),
            jax.ShapeDtypeStruct((128, 1), jnp.float32),
        ],
    )(W2.T, cvec(b2), cvec(g1), cvec(be1), s1, q1, h1)

    mx, mn, s3, q3 = pl.pallas_call(
        functools.partial(_l3_body, n_total=float(S)),
        grid=(_GROUP_SIZE,),
        in_specs=[
            pl.BlockSpec((_HIDDEN, 128), lambda j: (0, 0)),
            pl.BlockSpec((_HIDDEN, 1), lambda j: (0, 0)),
            pl.BlockSpec((128, 1), lambda j: (0, 0)),
            pl.BlockSpec((128, 1), lambda j: (0, 0)),
            pl.BlockSpec((128, 1), lambda j: (0, 0)),
            pl.BlockSpec((128, 1), lambda j: (0, 0)),
            pl.BlockSpec((128, NG), lambda j: (0, j)),
        ],
        out_specs=[
            pl.BlockSpec((_HIDDEN, NG), lambda j: (0, 0)),
            pl.BlockSpec((_HIDDEN, NG), lambda j: (0, 0)),
            pl.BlockSpec((_HIDDEN, 1), lambda j: (0, 0)),
            pl.BlockSpec((_HIDDEN, 1), lambda j: (0, 0)),
        ],
        out_shape=[
            jax.ShapeDtypeStruct((_HIDDEN, NG), jnp.float32),
            jax.ShapeDtypeStruct((_HIDDEN, NG), jnp.float32),
            jax.ShapeDtypeStruct((_HIDDEN, 1), jnp.float32),
            jax.ShapeDtypeStruct((_HIDDEN, 1), jnp.float32),
        ],
    )(W3.T, cvec(b3), cvec(g2), cvec(be2), s2, q2, h2)

    tok_t = pl.pallas_call(
        functools.partial(_fin_body, n_total=float(S)),
        out_shape=jax.ShapeDtypeStruct((_HIDDEN, NG), jnp.float32),
    )(cvec(g3), cvec(be3), s3, q3, mx, mn)
    return tok_t


_NTILE = 32
_CPT = 64
_N = 8192
_NCHUNK = _N // 16


def _knn_sc_body(pts_hbm, cen_hbm, out_hbm,
                 X, Y, Z, CX, CY, CZ, d2i, subhist, histc, sel,
                 cmins, qlist, cua, cia, cub, cib, outbuf):
    NC = 2
    wid = lax.axis_index("s") * NC + lax.axis_index("c")
    b = wid // 4
    mb = (wid % 4) * _CPT

    pltpu.sync_copy(pts_hbm.at[pl.ds((b * 3 + 0) * _N, _N)], X)
    pltpu.sync_copy(pts_hbm.at[pl.ds((b * 3 + 1) * _N, _N)], Y)
    pltpu.sync_copy(pts_hbm.at[pl.ds((b * 3 + 2) * _N, _N)], Z)
    pltpu.sync_copy(cen_hbm.at[pl.ds((b * 3 + 0) * _NUM_GROUPS + mb, _CPT)],
                    CX.at[pl.ds(0, _CPT)])
    pltpu.sync_copy(cen_hbm.at[pl.ds((b * 3 + 1) * _NUM_GROUPS + mb, _CPT)],
                    CY.at[pl.ds(0, _CPT)])
    pltpu.sync_copy(cen_hbm.at[pl.ds((b * 3 + 2) * _NUM_GROUPS + mb, _CPT)],
                    CZ.at[pl.ds(0, _CPT)])

    MAXI = jnp.int32(0x7FFFFFFF)
    iota16 = lax.iota(jnp.int32, 16)
    ones16 = jnp.ones((16,), jnp.int32)
    zeros16 = jnp.zeros((16,), jnp.int32)
    lanebase = iota16 * 256

    def zero_subhist():
        @plsc.parallel_loop(0, 16)
        def _z(t):
            for l in range(16):
                subhist[pl.ds(t * 256 + l * 16, 16)] = zeros16

    def reduce_hist():
        @plsc.parallel_loop(0, 16)
        def _red(t):
            s = zeros16
            for l in range(16):
                s = s + subhist[pl.ds(l * 256 + t * 16, 16)]
            histc[pl.ds(t * 16, 16)] = s

    def find_threshold(k_rem):
        def th(t, carry):
            base, T = carry
            h = histc[pl.ds(t * 16, 16)]
            cums = plsc.cumsum(h) + base
            glane = iota16 + t * 16
            candT = jnp.min(jnp.where(cums >= k_rem, glane, 9999))
            return base + jnp.sum(h), jnp.minimum(T, candT)
        _, T = lax.fori_loop(0, 16, th, (jnp.int32(0), jnp.int32(9999)))
        return T

    def row(m, carry0):
        msplat = jnp.full((16,), m, jnp.int32)
        cxv = plsc.load_gather(CX, [msplat])
        cyv = plsc.load_gather(CY, [msplat])
        czv = plsc.load_gather(CZ, [msplat])

        zero_subhist()

        @plsc.parallel_loop(0, _NCHUNK // 16)
        def _p1(g):
            acc = jnp.full((16,), MAXI, jnp.int32)
            for dt in range(16):
                off = (g * 16 + dt) * 16
                dx = X[pl.ds(off, 16)] - cxv
                dy = Y[pl.ds(off, 16)] - cyv
                dz = Z[pl.ds(off, 16)] - czv
                d2 = dx * dx + dy * dy + dz * dz
                u = plsc.bitcast(d2, jnp.int32)
                d2i[pl.ds(off, 16)] = u
                bkt = lax.shift_right_logical(u, 23)
                plsc.addupdate_scatter(subhist, [lanebase + bkt], ones16)
                acc = jnp.where(iota16 == dt, jnp.min(u), acc)
            cmins[pl.ds(g * 16, 16)] = acc

        reduce_hist()
        T1 = find_threshold(jnp.int32(_GROUP_SIZE))

        qb = jnp.where(T1 >= 255, MAXI, (T1 + 1) << 23)

        @plsc.parallel_loop(0, _NCHUNK // 16, carry=jnp.int32(0))
        def nq(t, nqc):
            v = cmins[pl.ds(t * 16, 16)]
            cidv = iota16 + t * 16
            maskq = v < qb
            plsc.store_compressed(qlist.at[pl.ds(nqc, 16)], cidv, mask=maskq)
            return nqc + jnp.sum(maskq.astype(jnp.int32))

        def cq(t, carry):
            selcnt, nout = carry
            tsp = jnp.full((16,), t, jnp.int32)
            cid = jnp.max(plsc.load_gather(qlist, [tsp]))
            u = d2i[pl.ds(cid * 16, 16)]
            glane = iota16 + cid * 16
            bkt = lax.shift_right_logical(u, 23)
            selm = bkt < T1
            eqm = bkt == T1
            plsc.store_compressed(sel.at[pl.ds(selcnt, 16)], glane, mask=selm)
            plsc.store_compressed(cua.at[pl.ds(nout, 16)], u, mask=eqm)
            plsc.store_compressed(cia.at[pl.ds(nout, 16)], glane, mask=eqm)
            return (jnp.minimum(selcnt + jnp.sum(selm.astype(jnp.int32)),
                                jnp.int32(32)),
                    jnp.minimum(nout + jnp.sum(eqm.astype(jnp.int32)),
                                jnp.int32(_N)))
        selcnt, ncand = plsc.parallel_loop(
            0, nq, carry=(jnp.int32(0), jnp.int32(0)))(
            lambda t, c: cq(t, c))

        bufs = [(cua, cia, cub, cib), (cub, cib, cua, cia), (cua, cia, cub, cib)]
        for (inu, ini, outu, outi), shift in zip(bufs, (15, 7, 0)):
            k_rem = _GROUP_SIZE - selcnt
            nch = (ncand + 15) >> 4
            zero_subhist()

            def h2(t, c, inu=inu, shift=shift, ncand=ncand):
                u = inu[pl.ds(t * 16, 16)]
                glane = iota16 + t * 16
                valid = glane < ncand
                bkt = lax.shift_right_logical(u, shift) & 0xFF
                plsc.addupdate_scatter(subhist, [lanebase + bkt], ones16,
                                       mask=valid)
                return c
            lax.fori_loop(0, nch, h2, 0)

            reduce_hist()
            T = find_threshold(k_rem)

            def cp(t, carry, inu=inu, ini=ini, outu=outu, outi=outi,
                   shift=shift, ncand=ncand, T=T):
                scnt, nout = carry
                u = inu[pl.ds(t * 16, 16)]
                ii = ini[pl.ds(t * 16, 16)]
                glane = iota16 + t * 16
                valid = glane < ncand
                bkt = lax.shift_right_logical(u, shift) & 0xFF
                selm = valid & (bkt < T)
                eqm = valid & (bkt == T)
                plsc.store_compressed(sel.at[pl.ds(scnt, 16)], ii, mask=selm)
                plsc.store_compressed(outu.at[pl.ds(nout, 16)], u, mask=eqm)
                plsc.store_compressed(outi.at[pl.ds(nout, 16)], ii, mask=eqm)
                return (jnp.minimum(scnt + jnp.sum(selm.astype(jnp.int32)),
                                    jnp.int32(32)),
                        jnp.minimum(nout + jnp.sum(eqm.astype(jnp.int32)),
                                    jnp.int32(_N)))
            selcnt, ncand = lax.fori_loop(0, nch, cp, (selcnt, jnp.int32(0)))

        k_rem = _GROUP_SIZE - selcnt

        def fill(t, scnt):
            ii = cib[pl.ds(t * 16, 16)]
            glane = iota16 + t * 16
            maskf = glane < k_rem
            plsc.store_compressed(sel.at[pl.ds(scnt, 16)], ii, mask=maskf)
            return scnt + jnp.sum(maskf.astype(jnp.int32))
        selcnt = lax.fori_loop(0, 2, fill, selcnt)

        @plsc.parallel_loop(0, 2)
        def _gat(c):
            idxv = sel[pl.ds(c * 16, 16)]
            jv = iota16 + c * 16
            px = plsc.load_gather(X, [idxv]) - cxv
            py = plsc.load_gather(Y, [idxv]) - cyv
            pz = plsc.load_gather(Z, [idxv]) - czv
            zsp = jnp.zeros((16,), jnp.int32)
            plsc.store_scatter(outbuf, [zsp, jv, msplat], px)
            plsc.store_scatter(outbuf, [zsp + 1, jv, msplat], py)
            plsc.store_scatter(outbuf, [zsp + 2, jv, msplat], pz)
        return carry0

    lax.fori_loop(0, _CPT, row, 0)
    pltpu.sync_copy(outbuf, out_hbm.at[wid])


def _knn_sc(points, cen_t):
    B = points.shape[0]
    pts_flat = points.transpose(0, 2, 1).reshape(B * 3 * _N)
    cen_flat = cen_t.transpose(1, 0, 2).reshape(B * 3 * _NUM_GROUPS)
    mesh = plsc.VectorSubcoreMesh(core_axis_name="c", subcore_axis_name="s")
    f32, i32 = jnp.float32, jnp.int32
    out = pl.kernel(
        _knn_sc_body,
        mesh=mesh,
        out_type=jax.ShapeDtypeStruct((_NTILE, 3, _GROUP_SIZE, _CPT), f32),
        compiler_params=pltpu.CompilerParams(needs_layout_passes=False),
        scratch_types=[
            pltpu.VMEM((_N,), f32), pltpu.VMEM((_N,), f32),
            pltpu.VMEM((_N,), f32),
            pltpu.VMEM((128,), f32), pltpu.VMEM((128,), f32),
            pltpu.VMEM((128,), f32),
            pltpu.VMEM((_N,), i32),
            pltpu.VMEM((4096,), i32),
            pltpu.VMEM((256,), i32),
            pltpu.VMEM((48,), i32),
            pltpu.VMEM((_NCHUNK,), i32),
            pltpu.VMEM((_NCHUNK + 16,), i32),
            pltpu.VMEM((_N + 16,), i32), pltpu.VMEM((_N + 16,), i32),
            pltpu.VMEM((_N + 16,), i32), pltpu.VMEM((_N + 16,), i32),
            pltpu.VMEM((3, _GROUP_SIZE, _CPT), f32),
        ],
    )(pts_flat, cen_flat)
    G = out.transpose(1, 2, 0, 3).reshape(3, _GROUP_SIZE * B * _NUM_GROUPS)
    return G


def kernel(points, W1, b1, g1, be1, W2, b2, g2, be2, W3, b3, g3, be3):
    B, N, _ = points.shape
    cen_t = _fps(points)
    centers = cen_t.transpose(1, 2, 0)

    G = _knn_sc(points, cen_t)

    tok_t = _pointnet(G, W1, b1, g1, be1, W2, b2, g2, be2, W3, b3, g3, be3)
    tokens = tok_t.T.reshape(B, _NUM_GROUPS, _HIDDEN)
    return (tokens, centers)


# speedup vs baseline: 2.0336x; 1.0036x over previous
import functools

import jax
import jax.numpy as jnp
from jax import lax
from jax.experimental import pallas as pl
from jax.experimental.pallas import tpu as pltpu
from jax.experimental.pallas import tpu_sc as plsc

_NUM_GROUPS = 256
_GROUP_SIZE = 32
_HIDDEN = 384
_EPS = 1e-5


def _fps_body(pts_ref, f0_ref, cen_ref):
    X = pts_ref[0]
    Y = pts_ref[1]
    Z = pts_ref[2]
    B, N = X.shape
    col = jax.lax.broadcasted_iota(jnp.int32, (B, N), 1)
    colc = jax.lax.broadcasted_iota(jnp.int32, (B, _NUM_GROUPS), 1)
    cen_ref[...] = jnp.zeros_like(cen_ref)

    def body(i, carry):
        dist, f = carry
        onehot = col == f
        cx = jnp.sum(jnp.where(onehot, X, 0.0), axis=1, keepdims=True)
        cy = jnp.sum(jnp.where(onehot, Y, 0.0), axis=1, keepdims=True)
        cz = jnp.sum(jnp.where(onehot, Z, 0.0), axis=1, keepdims=True)
        sel = colc == i
        cen_ref[0] = jnp.where(sel, cx, cen_ref[0])
        cen_ref[1] = jnp.where(sel, cy, cen_ref[1])
        cen_ref[2] = jnp.where(sel, cz, cen_ref[2])
        d = (X - cx) ** 2 + (Y - cy) ** 2 + (Z - cz) ** 2
        dist = jnp.minimum(dist, d)
        m = jnp.max(dist, axis=1, keepdims=True)
        f = jnp.min(jnp.where(dist == m, col, N), axis=1, keepdims=True)
        return dist, f

    dist0 = jnp.full((B, N), 1e10, dtype=jnp.float32)
    f0 = f0_ref[...]
    jax.lax.fori_loop(0, _NUM_GROUPS, body, (dist0, f0))


def _fps(points):
    B, N, _ = points.shape
    pts_t = points.transpose(2, 0, 1)
    f0 = jax.random.randint(jax.random.key(42), (B,), 0, N).astype(jnp.int32)
    cen_t = pl.pallas_call(
        _fps_body,
        out_shape=jax.ShapeDtypeStruct((3, B, _NUM_GROUPS), jnp.float32),
    )(pts_t, f0.reshape(B, 1))
    return cen_t


def _bn_affine(g_ref, be_ref, s_ref, q_ref, n_total):
    mean = s_ref[...] / n_total
    var = q_ref[...] / n_total - mean * mean
    s = g_ref[...] * jax.lax.rsqrt(var + _EPS)
    t = be_ref[...] - mean * s
    return s, t


def _sa_body(w1t_ref, b1_ref, g_ref, ssum_ref, ssq_ref):
    i = pl.program_id(0)
    h = jnp.dot(w1t_ref[...], g_ref[...], preferred_element_type=jnp.float32)
    h = h + b1_ref[...]

    @pl.when(i == 0)
    def _():
        ssum_ref[...] = jnp.zeros_like(ssum_ref)
        ssq_ref[...] = jnp.zeros_like(ssq_ref)

    ssum_ref[...] += jnp.sum(h, axis=1, keepdims=True)
    ssq_ref[...] += jnp.sum(h * h, axis=1, keepdims=True)


def _sb_body(w1t_ref, b1_ref, g1_ref, be1_ref, s1_ref, q1_ref,
             w2t_ref, b2_ref, g_ref, ssum_ref, ssq_ref, *, n_total):
    i = pl.program_id(0)
    h1 = jnp.dot(w1t_ref[...], g_ref[...], preferred_element_type=jnp.float32)
    h1 = h1 + b1_ref[...]
    s, t = _bn_affine(g1_ref, be1_ref, s1_ref, q1_ref, n_total)
    a1 = jax.nn.relu(s * h1 + t)
    h2 = jnp.dot(w2t_ref[...], a1, preferred_element_type=jnp.float32)
    h2 = h2 + b2_ref[...]

    @pl.when(i == 0)
    def _():
        ssum_ref[...] = jnp.zeros_like(ssum_ref)
        ssq_ref[...] = jnp.zeros_like(ssq_ref)

    ssum_ref[...] += jnp.sum(h2, axis=1, keepdims=True)
    ssq_ref[...] += jnp.sum(h2 * h2, axis=1, keepdims=True)


def _sc_body(w1t_ref, b1_ref, g1_ref, be1_ref, s1_ref, q1_ref,
             w2t_ref, b2_ref, g2_ref, be2_ref, s2_ref, q2_ref,
             w3t_ref, b3_ref, g_ref,
             mx_ref, mn_ref, ssum_ref, ssq_ref, *, n_total):
    j = pl.program_id(0)
    h1 = jnp.dot(w1t_ref[...], g_ref[...], preferred_element_type=jnp.float32)
    h1 = h1 + b1_ref[...]
    s, t = _bn_affine(g1_ref, be1_ref, s1_ref, q1_ref, n_total)
    a1 = jax.nn.relu(s * h1 + t)
    h2 = jnp.dot(w2t_ref[...], a1, preferred_element_type=jnp.float32)
    h2 = h2 + b2_ref[...]
    s2, t2 = _bn_affine(g2_ref, be2_ref, s2_ref, q2_ref, n_total)
    a2 = jax.nn.relu(s2 * h2 + t2)
    h = jnp.dot(w3t_ref[...], a2, preferred_element_type=jnp.float32)
    h = h + b3_ref[...]

    @pl.when(j == 0)
    def _():
        ssum_ref[...] = jnp.zeros_like(ssum_ref)
        ssq_ref[...] = jnp.zeros_like(ssq_ref)
        mx_ref[...] = jnp.full_like(mx_ref, -jnp.inf)
        mn_ref[...] = jnp.full_like(mn_ref, jnp.inf)

    ssum_ref[...] += jnp.sum(h, axis=1, keepdims=True)
    ssq_ref[...] += jnp.sum(h * h, axis=1, keepdims=True)
    mx_ref[...] = jnp.maximum(mx_ref[...], h)
    mn_ref[...] = jnp.minimum(mn_ref[...], h)


def _fin_body(g3_ref, be3_ref, s3_ref, q3_ref, mx_ref, mn_ref, tok_ref, *, n_total):
    s, t = _bn_affine(g3_ref, be3_ref, s3_ref, q3_ref, n_total)
    picked = jnp.where(s >= 0.0, mx_ref[...], mn_ref[...])
    tok_ref[...] = picked * s + t


def _pointnet(G, W1, b1, g1, be1, W2, b2, g2, be2, W3, b3, g3, be3):
    S = G.shape[1]
    NG = S // _GROUP_SIZE
    cvec = lambda v: v.reshape(-1, 1)
    cp = lambda r, c: pl.BlockSpec((r, c), lambda i: (0, 0))

    LBLK = 16384
    nblk = S // LBLK
    s1, q1 = pl.pallas_call(
        _sa_body,
        grid=(nblk,),
        in_specs=[cp(64, 3), cp(64, 1), pl.BlockSpec((3, LBLK), lambda i: (0, i))],
        out_specs=[cp(64, 1), cp(64, 1)],
        out_shape=[jax.ShapeDtypeStruct((64, 1), jnp.float32)] * 2,
    )(W1.T, cvec(b1), G)

    s2, q2 = pl.pallas_call(
        functools.partial(_sb_body, n_total=float(S)),
        grid=(nblk,),
        in_specs=[cp(64, 3), cp(64, 1), cp(64, 1), cp(64, 1), cp(64, 1),
                  cp(64, 1), cp(128, 64), cp(128, 1),
                  pl.BlockSpec((3, LBLK), lambda i: (0, i))],
        out_specs=[cp(128, 1), cp(128, 1)],
        out_shape=[jax.ShapeDtypeStruct((128, 1), jnp.float32)] * 2,
    )(W1.T, cvec(b1), cvec(g1), cvec(be1), s1, q1, W2.T, cvec(b2), G)

    mx, mn, s3, q3 = pl.pallas_call(
        functools.partial(_sc_body, n_total=float(S)),
        grid=(_GROUP_SIZE,),
        in_specs=[cp(64, 3), cp(64, 1), cp(64, 1), cp(64, 1), cp(64, 1),
                  cp(64, 1), cp(128, 64), cp(128, 1), cp(128, 1), cp(128, 1),
                  cp(128, 1), cp(128, 1), cp(_HIDDEN, 128), cp(_HIDDEN, 1),
                  pl.BlockSpec((3, NG), lambda j: (0, j))],
        out_specs=[cp(_HIDDEN, NG), cp(_HIDDEN, NG), cp(_HIDDEN, 1),
                   cp(_HIDDEN, 1)],
        out_shape=[
            jax.ShapeDtypeStruct((_HIDDEN, NG), jnp.float32),
            jax.ShapeDtypeStruct((_HIDDEN, NG), jnp.float32),
            jax.ShapeDtypeStruct((_HIDDEN, 1), jnp.float32),
            jax.ShapeDtypeStruct((_HIDDEN, 1), jnp.float32),
        ],
    )(W1.T, cvec(b1), cvec(g1), cvec(be1), s1, q1, W2.T, cvec(b2),
      cvec(g2), cvec(be2), s2, q2, W3.T, cvec(b3), G)

    tok_t = pl.pallas_call(
        functools.partial(_fin_body, n_total=float(S)),
        out_shape=jax.ShapeDtypeStruct((_HIDDEN, NG), jnp.float32),
    )(cvec(g3), cvec(be3), s3, q3, mx, mn)
    return tok_t


_NTILE = 32
_CPT = 64
_N = 8192
_NCHUNK = _N // 16


def _knn_sc_body(pts_hbm, cen_hbm, out_hbm,
                 X, Y, Z, CX, CY, CZ, d2i, subhist, histc, sel,
                 cmins, qlist, cua, cia, cub, cib, outbuf):
    NC = 2
    wid = lax.axis_index("s") * NC + lax.axis_index("c")
    b = wid // 4
    mb = (wid % 4) * _CPT

    pltpu.sync_copy(pts_hbm.at[pl.ds((b * 3 + 0) * _N, _N)], X)
    pltpu.sync_copy(pts_hbm.at[pl.ds((b * 3 + 1) * _N, _N)], Y)
    pltpu.sync_copy(pts_hbm.at[pl.ds((b * 3 + 2) * _N, _N)], Z)
    pltpu.sync_copy(cen_hbm.at[pl.ds((b * 3 + 0) * _NUM_GROUPS + mb, _CPT)],
                    CX.at[pl.ds(0, _CPT)])
    pltpu.sync_copy(cen_hbm.at[pl.ds((b * 3 + 1) * _NUM_GROUPS + mb, _CPT)],
                    CY.at[pl.ds(0, _CPT)])
    pltpu.sync_copy(cen_hbm.at[pl.ds((b * 3 + 2) * _NUM_GROUPS + mb, _CPT)],
                    CZ.at[pl.ds(0, _CPT)])

    MAXI = jnp.int32(0x7FFFFFFF)
    iota16 = lax.iota(jnp.int32, 16)
    ones16 = jnp.ones((16,), jnp.int32)
    zeros16 = jnp.zeros((16,), jnp.int32)
    lanebase = iota16 * 256

    def zero_subhist():
        @plsc.parallel_loop(0, 16)
        def _z(t):
            for l in range(16):
                subhist[pl.ds(t * 256 + l * 16, 16)] = zeros16

    def reduce_hist():
        @plsc.parallel_loop(0, 16)
        def _red(t):
            s = zeros16
            for l in range(16):
                s = s + subhist[pl.ds(l * 256 + t * 16, 16)]
            histc[pl.ds(t * 16, 16)] = s

    def find_threshold(k_rem):
        def th(t, carry):
            base, T = carry
            h = histc[pl.ds(t * 16, 16)]
            cums = plsc.cumsum(h) + base
            glane = iota16 + t * 16
            candT = jnp.min(jnp.where(cums >= k_rem, glane, 9999))
            return base + jnp.sum(h), jnp.minimum(T, candT)
        _, T = lax.fori_loop(0, 16, th, (jnp.int32(0), jnp.int32(9999)))
        return T

    def row(m, carry0):
        msplat = jnp.full((16,), m, jnp.int32)
        cxv = plsc.load_gather(CX, [msplat])
        cyv = plsc.load_gather(CY, [msplat])
        czv = plsc.load_gather(CZ, [msplat])

        zero_subhist()

        @plsc.parallel_loop(0, _NCHUNK // 16)
        def _p1(g):
            acc = jnp.full((16,), MAXI, jnp.int32)
            for dt in range(16):
                off = (g * 16 + dt) * 16
                dx = X[pl.ds(off, 16)] - cxv
                dy = Y[pl.ds(off, 16)] - cyv
                dz = Z[pl.ds(off, 16)] - czv
                d2 = dx * dx + dy * dy + dz * dz
                u = plsc.bitcast(d2, jnp.int32)
                d2i[pl.ds(off, 16)] = u
                bkt = lax.shift_right_logical(u, 23)
                plsc.addupdate_scatter(subhist, [lanebase + bkt], ones16)
                acc = jnp.where(iota16 == dt, jnp.min(u), acc)
            cmins[pl.ds(g * 16, 16)] = acc

        reduce_hist()
        T1 = find_threshold(jnp.int32(_GROUP_SIZE))

        qb = jnp.where(T1 >= 255, MAXI, (T1 + 1) << 23)

        @plsc.parallel_loop(0, _NCHUNK // 16, carry=jnp.int32(0))
        def nq(t, nqc):
            v = cmins[pl.ds(t * 16, 16)]
            cidv = iota16 + t * 16
            maskq = v < qb
            plsc.store_compressed(qlist.at[pl.ds(nqc, 16)], cidv, mask=maskq)
            return nqc + jnp.sum(maskq.astype(jnp.int32))

        def cq(t, carry):
            selcnt, nout = carry
            tsp = jnp.full((16,), t, jnp.int32)
            cid = jnp.max(plsc.load_gather(qlist, [tsp]))
            u = d2i[pl.ds(cid * 16, 16)]
            glane = iota16 + cid * 16
            bkt = lax.shift_right_logical(u, 23)
            selm = bkt < T1
            eqm = bkt == T1
            plsc.store_compressed(sel.at[pl.ds(selcnt, 16)], glane, mask=selm)
            plsc.store_compressed(cua.at[pl.ds(nout, 16)], u, mask=eqm)
            plsc.store_compressed(cia.at[pl.ds(nout, 16)], glane, mask=eqm)
            return (jnp.minimum(selcnt + jnp.sum(selm.astype(jnp.int32)),
                                jnp.int32(32)),
                    jnp.minimum(nout + jnp.sum(eqm.astype(jnp.int32)),
                                jnp.int32(_N)))
        selcnt, ncand = plsc.parallel_loop(
            0, nq, carry=(jnp.int32(0), jnp.int32(0)))(
            lambda t, c: cq(t, c))

        bufs = [(cua, cia, cub, cib), (cub, cib, cua, cia), (cua, cia, cub, cib)]
        for (inu, ini, outu, outi), shift in zip(bufs, (15, 7, 0)):
            k_rem = _GROUP_SIZE - selcnt
            nch = (ncand + 15) >> 4
            zero_subhist()

            def h2(t, c, inu=inu, shift=shift, ncand=ncand):
                u = inu[pl.ds(t * 16, 16)]
                glane = iota16 + t * 16
                valid = glane < ncand
                bkt = lax.shift_right_logical(u, shift) & 0xFF
                plsc.addupdate_scatter(subhist, [lanebase + bkt], ones16,
                                       mask=valid)
                return c
            lax.fori_loop(0, nch, h2, 0)

            reduce_hist()
            T = find_threshold(k_rem)

            def cp(t, carry, inu=inu, ini=ini, outu=outu, outi=outi,
                   shift=shift, ncand=ncand, T=T):
                scnt, nout = carry
                u = inu[pl.ds(t * 16, 16)]
                ii = ini[pl.ds(t * 16, 16)]
                glane = iota16 + t * 16
                valid = glane < ncand
                bkt = lax.shift_right_logical(u, shift) & 0xFF
                selm = valid & (bkt < T)
                eqm = valid & (bkt == T)
                plsc.store_compressed(sel.at[pl.ds(scnt, 16)], ii, mask=selm)
                plsc.store_compressed(outu.at[pl.ds(nout, 16)], u, mask=eqm)
                plsc.store_compressed(outi.at[pl.ds(nout, 16)], ii, mask=eqm)
                return (jnp.minimum(scnt + jnp.sum(selm.astype(jnp.int32)),
                                    jnp.int32(32)),
                        jnp.minimum(nout + jnp.sum(eqm.astype(jnp.int32)),
                                    jnp.int32(_N)))
            selcnt, ncand = lax.fori_loop(0, nch, cp, (selcnt, jnp.int32(0)))

        k_rem = _GROUP_SIZE - selcnt

        def fill(t, scnt):
            ii = cib[pl.ds(t * 16, 16)]
            glane = iota16 + t * 16
            maskf = glane < k_rem
            plsc.store_compressed(sel.at[pl.ds(scnt, 16)], ii, mask=maskf)
            return scnt + jnp.sum(maskf.astype(jnp.int32))
        selcnt = lax.fori_loop(0, 2, fill, selcnt)

        @plsc.parallel_loop(0, 2)
        def _gat(c):
            idxv = sel[pl.ds(c * 16, 16)]
            jv = iota16 + c * 16
            px = plsc.load_gather(X, [idxv]) - cxv
            py = plsc.load_gather(Y, [idxv]) - cyv
            pz = plsc.load_gather(Z, [idxv]) - czv
            zsp = jnp.zeros((16,), jnp.int32)
            plsc.store_scatter(outbuf, [zsp, jv, msplat], px)
            plsc.store_scatter(outbuf, [zsp + 1, jv, msplat], py)
            plsc.store_scatter(outbuf, [zsp + 2, jv, msplat], pz)
        return carry0

    lax.fori_loop(0, _CPT, row, 0)
    pltpu.sync_copy(outbuf, out_hbm.at[wid])


def _knn_sc(points, cen_t):
    B = points.shape[0]
    pts_flat = points.transpose(0, 2, 1).reshape(B * 3 * _N)
    cen_flat = cen_t.transpose(1, 0, 2).reshape(B * 3 * _NUM_GROUPS)
    mesh = plsc.VectorSubcoreMesh(core_axis_name="c", subcore_axis_name="s")
    f32, i32 = jnp.float32, jnp.int32
    out = pl.kernel(
        _knn_sc_body,
        mesh=mesh,
        out_type=jax.ShapeDtypeStruct((_NTILE, 3, _GROUP_SIZE, _CPT), f32),
        compiler_params=pltpu.CompilerParams(needs_layout_passes=False),
        scratch_types=[
            pltpu.VMEM((_N,), f32), pltpu.VMEM((_N,), f32),
            pltpu.VMEM((_N,), f32),
            pltpu.VMEM((128,), f32), pltpu.VMEM((128,), f32),
            pltpu.VMEM((128,), f32),
            pltpu.VMEM((_N,), i32),
            pltpu.VMEM((4096,), i32),
            pltpu.VMEM((256,), i32),
            pltpu.VMEM((48,), i32),
            pltpu.VMEM((_NCHUNK,), i32),
            pltpu.VMEM((_NCHUNK + 16,), i32),
            pltpu.VMEM((_N + 16,), i32), pltpu.VMEM((_N + 16,), i32),
            pltpu.VMEM((_N + 16,), i32), pltpu.VMEM((_N + 16,), i32),
            pltpu.VMEM((3, _GROUP_SIZE, _CPT), f32),
        ],
    )(pts_flat, cen_flat)
    G = out.transpose(1, 2, 0, 3).reshape(3, _GROUP_SIZE * B * _NUM_GROUPS)
    return G


def kernel(points, W1, b1, g1, be1, W2, b2, g2, be2, W3, b3, g3, be3):
    B, N, _ = points.shape
    cen_t = _fps(points)
    centers = cen_t.transpose(1, 2, 0)

    G = _knn_sc(points, cen_t)

    tok_t = _pointnet(G, W1, b1, g1, be1, W2, b2, g2, be2, W3, b3, g3, be3)
    tokens = tok_t.T.reshape(B, _NUM_GROUPS, _HIDDEN)
    return (tokens, centers)
